# baseline (speedup 1.0000x reference)
import functools
import jax
import jax.numpy as jnp
from jax.experimental import pallas as pl
from jax.experimental.pallas import tpu as pltpu
from jax.experimental.pallas import tpu_sc as plsc

T = 2048
D = 1024
NH = 16
NKV = 4
HD = 64
E = 8
TOPK = 2
NG = 4
TG = 2
DFF = 512
EPS = 1e-05

BT = 256
NBT = T // BT
BQ = 512
NBQ = T // BQ

BTM = 256
NRT = 24
NPAD = NRT * BTM
NWORK = 32
TPW = T // NWORK

_HIGH = jax.lax.Precision.HIGHEST


def _rms(x, w, eps=EPS):
    v = jnp.mean(x * x, axis=-1, keepdims=True)
    return x * jax.lax.rsqrt(v + eps) * w


def _dot_nt(a, b):
    return jax.lax.dot_general(
        a.astype(jnp.bfloat16), b.astype(jnp.bfloat16),
        (((1,), (1,)), ((), ())), preferred_element_type=jnp.float32)


def _k1_body(h_ref, ln1_ref, qkvw_ref, gatew_ref, qn_ref, kn_ref,
             q_out, k_out, v_out, gate_out):
    x = h_ref[...]
    xn = _rms(x, ln1_ref[...])
    y = _dot_nt(xn, qkvw_ref[...])
    gate_out[...] = _dot_nt(xn, gatew_ref[...])
    qn = qn_ref[...]
    kn = kn_ref[...]
    for h in range(NH):
        qh = _rms(y[:, h * HD:(h + 1) * HD], qn) * (HD ** -0.5)
        q_out[h] = qh.astype(jnp.bfloat16)
    for h in range(NKV):
        kb = NH * HD + h * HD
        k_out[h] = _rms(y[:, kb:kb + HD], kn).astype(jnp.bfloat16)
        vb = (NH + NKV) * HD + h * HD
        v_out[h] = y[:, vb:vb + HD].astype(jnp.bfloat16)


def _k1(h, ln1_w, qkv_w, attn_gate_w, q_norm_w, k_norm_w):
    return pl.pallas_call(
        _k1_body,
        grid=(NBT,),
        in_specs=[
            pl.BlockSpec((BT, D), lambda i: (i, 0)),
            pl.BlockSpec((1, D), lambda i: (0, 0)),
            pl.BlockSpec(((NH + 2 * NKV) * HD, D), lambda i: (0, 0)),
            pl.BlockSpec((NH * HD, D), lambda i: (0, 0)),
            pl.BlockSpec((1, HD), lambda i: (0, 0)),
            pl.BlockSpec((1, HD), lambda i: (0, 0)),
        ],
        out_specs=[
            pl.BlockSpec((NH, BT, HD), lambda i: (0, i, 0)),
            pl.BlockSpec((NKV, BT, HD), lambda i: (0, i, 0)),
            pl.BlockSpec((NKV, BT, HD), lambda i: (0, i, 0)),
            pl.BlockSpec((BT, NH * HD), lambda i: (i, 0)),
        ],
        out_shape=[
            jax.ShapeDtypeStruct((NH, T, HD), jnp.bfloat16),
            jax.ShapeDtypeStruct((NKV, T, HD), jnp.bfloat16),
            jax.ShapeDtypeStruct((NKV, T, HD), jnp.bfloat16),
            jax.ShapeDtypeStruct((T, NH * HD), jnp.float32),
        ],
    )(h, ln1_w.reshape(1, D), qkv_w, attn_gate_w,
      q_norm_w.reshape(1, HD), k_norm_w.reshape(1, HD))


def _attn_body(q_ref, k_ref, v_ref, o_ref):
    i = pl.program_id(1)
    qb = q_ref[0]

    def pexp(s):
        return jnp.exp(jnp.minimum(s, 30.0))

    def body(j, carry):
        l, acc = carry
        kb = k_ref[0, pl.ds(j * BQ, BQ), :]
        vb = v_ref[0, pl.ds(j * BQ, BQ), :]
        s = jax.lax.dot_general(qb, kb, (((1,), (1,)), ((), ())),
                                preferred_element_type=jnp.float32)
        p = pexp(s)
        l = l + jnp.sum(p, axis=-1, keepdims=True)
        acc = acc + jax.lax.dot_general(
            p.astype(jnp.bfloat16), vb, (((1,), (0,)), ((), ())),
            preferred_element_type=jnp.float32)
        return l, acc

    l0 = jnp.zeros((BQ, 1), jnp.float32)
    a0 = jnp.zeros((BQ, HD), jnp.float32)
    l, acc = jax.lax.fori_loop(0, i, body, (l0, a0))

    rows = jax.lax.broadcasted_iota(jnp.int32, (BQ, BQ), 0)
    cols = jax.lax.broadcasted_iota(jnp.int32, (BQ, BQ), 1)
    maskf = (cols <= rows).astype(jnp.float32)
    kb = k_ref[0, pl.ds(i * BQ, BQ), :]
    vb = v_ref[0, pl.ds(i * BQ, BQ), :]
    s = jax.lax.dot_general(qb, kb, (((1,), (1,)), ((), ())),
                            preferred_element_type=jnp.float32)
    p = pexp(s) * maskf
    l = l + jnp.sum(p, axis=-1, keepdims=True)
    acc = acc + jax.lax.dot_general(
        p.astype(jnp.bfloat16), vb, (((1,), (0,)), ((), ())),
        preferred_element_type=jnp.float32)

    o_ref[0] = acc / l


def _k2(q, k, v):
    rep = NH // NKV
    return pl.pallas_call(
        _attn_body,
        grid=(NH, NBQ),
        in_specs=[
            pl.BlockSpec((1, BQ, HD), lambda h, i: (h, i, 0)),
            pl.BlockSpec((1, T, HD), lambda h, i: (h // rep, 0, 0)),
            pl.BlockSpec((1, T, HD), lambda h, i: (h // rep, 0, 0)),
        ],
        out_specs=pl.BlockSpec((1, BQ, HD), lambda h, i: (h, i, 0)),
        out_shape=jax.ShapeDtypeStruct((NH, T, HD), jnp.float32),
    )(q, k, v)


def _rank_lt(m, kmax, n):
    rank = jnp.zeros_like(m)
    e_idx = jax.lax.broadcasted_iota(jnp.int32, m.shape, 1)
    for r in range(1, n):
        m_rot = jnp.concatenate([m[:, r:], m[:, :r]], axis=1)
        beat = (m_rot > m) | ((m_rot == m) & (e_idx >= n - r))
        rank = rank + beat.astype(jnp.float32)
    return (rank < kmax).astype(jnp.float32)


def _k3_body(o_ref, gate_ref, res_ref, ow_ref, ln2_ref, rw_ref, bias_ref,
             h2_out, hn2_out, cmb_out, rank_out, off_out, cnt_ref):
    i = pl.program_id(0)

    @pl.when(i == 0)
    def _init():
        cnt_ref[...] = jnp.zeros((1, E), jnp.float32)

    og = o_ref[...] * jax.nn.sigmoid(gate_ref[...])
    h2 = _dot_nt(og, ow_ref[...]) + res_ref[...]
    h2_out[...] = h2
    hn2 = _rms(h2, ln2_ref[...])
    hn2_out[...] = hn2

    logits = jax.lax.dot_general(
        hn2, rw_ref[...], (((1,), (1,)), ((), ())),
        preferred_element_type=jnp.float32, precision=_HIGH)
    scores = jax.nn.sigmoid(logits)
    sfc = scores + bias_ref[...]

    epg = E // NG
    pa = jax.lax.broadcasted_iota(jnp.int32, (E, NG), 0)
    pg = jax.lax.broadcasted_iota(jnp.int32, (E, NG), 1)
    pair = ((pa // epg) == pg).astype(jnp.float32)
    gs = jax.lax.dot_general(
        sfc, pair, (((1,), (0,)), ((), ())),
        preferred_element_type=jnp.float32, precision=_HIGH)

    sel_g = _rank_lt(gs, TG, NG)
    ge = jax.lax.broadcasted_iota(jnp.int32, (NG, E), 0)
    ee = jax.lax.broadcasted_iota(jnp.int32, (NG, E), 1)
    expand = (ge == (ee // epg)).astype(jnp.float32)
    mask_e = jax.lax.dot_general(
        sel_g, expand, (((1,), (0,)), ((), ())),
        preferred_element_type=jnp.float32, precision=_HIGH)

    masked = jnp.where(mask_e > 0.5, sfc, -1e30)
    sel_e = _rank_lt(masked, TOPK, E)
    w = scores * sel_e
    denom = jnp.sum(w, axis=-1, keepdims=True) + 1e-20
    cmb_out[...] = w / denom

    lr = jax.lax.broadcasted_iota(jnp.int32, (BT, BT), 0)
    lc = jax.lax.broadcasted_iota(jnp.int32, (BT, BT), 1)
    lt = (lc < lr).astype(jnp.float32)
    rank_blk = jax.lax.dot_general(
        lt, sel_e, (((1,), (0,)), ((), ())),
        preferred_element_type=jnp.float32, precision=_HIGH)
    rank_out[...] = rank_blk + cnt_ref[...]
    new_cnt = cnt_ref[...] + jnp.sum(sel_e, axis=0, keepdims=True)
    cnt_ref[...] = new_cnt
    padded = jnp.floor((new_cnt + (BTM - 1)) * (1.0 / BTM)) * BTM
    ea = jax.lax.broadcasted_iota(jnp.int32, (E, E), 0)
    eb = jax.lax.broadcasted_iota(jnp.int32, (E, E), 1)
    lt8 = (ea < eb).astype(jnp.float32)
    off_out[...] = jax.lax.dot_general(
        padded, lt8, (((1,), (0,)), ((), ())),
        preferred_element_type=jnp.float32, precision=_HIGH)


def _k3(o2, gate, res, o_w, ln2_w, router_w, expert_bias):
    return pl.pallas_call(
        _k3_body,
        grid=(NBT,),
        in_specs=[
            pl.BlockSpec((BT, D), lambda i: (i, 0)),
            pl.BlockSpec((BT, D), lambda i: (i, 0)),
            pl.BlockSpec((BT, D), lambda i: (i, 0)),
            pl.BlockSpec((D, D), lambda i: (0, 0)),
            pl.BlockSpec((1, D), lambda i: (0, 0)),
            pl.BlockSpec((E, D), lambda i: (0, 0)),
            pl.BlockSpec((1, E), lambda i: (0, 0)),
        ],
        out_specs=[
            pl.BlockSpec((BT, D), lambda i: (i, 0)),
            pl.BlockSpec((BT, D), lambda i: (i, 0)),
            pl.BlockSpec((BT, E), lambda i: (i, 0)),
            pl.BlockSpec((BT, E), lambda i: (i, 0)),
            pl.BlockSpec((1, E), lambda i: (0, 0)),
        ],
        out_shape=[
            jax.ShapeDtypeStruct((T, D), jnp.float32),
            jax.ShapeDtypeStruct((T, D), jnp.float32),
            jax.ShapeDtypeStruct((T, E), jnp.float32),
            jax.ShapeDtypeStruct((T, E), jnp.float32),
            jax.ShapeDtypeStruct((1, E), jnp.float32),
        ],
        scratch_shapes=[pltpu.VMEM((1, E), jnp.float32)],
    )(o2, gate, res, o_w, ln2_w.reshape(1, D), router_w,
      expert_bias.reshape(1, E))


def _k3b_body(rank_ref, cmb_ref, off_ref, p0_ref, p1_ref, w0_ref, w1_ref,
              te_ref):
    cmb = cmb_ref[...]
    sel = (cmb > 0).astype(jnp.float32)
    ea = jax.lax.broadcasted_iota(jnp.int32, (E, E), 0)
    eb = jax.lax.broadcasted_iota(jnp.int32, (E, E), 1)
    lt8 = (ea < eb).astype(jnp.float32)
    s_excl = jax.lax.dot_general(
        sel, lt8, (((1,), (0,)), ((), ())),
        preferred_element_type=jnp.float32, precision=_HIGH)
    off = off_ref[...]
    pos = off + rank_ref[...]
    is0 = sel * (s_excl == 0).astype(jnp.float32)
    is1 = sel * (s_excl == 1).astype(jnp.float32)
    p0_ref[...] = jnp.sum(pos * is0, axis=1, keepdims=True).astype(jnp.int32)
    p1_ref[...] = jnp.sum(pos * is1, axis=1, keepdims=True).astype(jnp.int32)
    w0_ref[...] = jnp.sum(cmb * is0, axis=1, keepdims=True)
    w1_ref[...] = jnp.sum(cmb * is1, axis=1, keepdims=True)

    jt = jax.lax.broadcasted_iota(jnp.int32, (1, NRT), 1) * BTM
    te = jnp.zeros((1, NRT), jnp.int32)
    for e in range(1, E):
        te = te + (jt >= off[:, e:e + 1].astype(jnp.int32)).astype(jnp.int32)
    te_ref[...] = te


def _k3b(rank, cmb, off):
    return pl.pallas_call(
        _k3b_body,
        grid=(NBT,),
        in_specs=[
            pl.BlockSpec((BT, E), lambda i: (i, 0)),
            pl.BlockSpec((BT, E), lambda i: (i, 0)),
            pl.BlockSpec((1, E), lambda i: (0, 0)),
        ],
        out_specs=[
            pl.BlockSpec((BT, 1), lambda i: (i, 0)),
            pl.BlockSpec((BT, 1), lambda i: (i, 0)),
            pl.BlockSpec((BT, 1), lambda i: (i, 0)),
            pl.BlockSpec((BT, 1), lambda i: (i, 0)),
            pl.BlockSpec((1, NRT), lambda i: (0, 0)),
        ],
        out_shape=[
            jax.ShapeDtypeStruct((T, 1), jnp.int32),
            jax.ShapeDtypeStruct((T, 1), jnp.int32),
            jax.ShapeDtypeStruct((T, 1), jnp.float32),
            jax.ShapeDtypeStruct((T, 1), jnp.float32),
            jax.ShapeDtypeStruct((1, NRT), jnp.int32),
        ],
    )(rank, cmb, off)


@functools.cache
def _sc_dispatch_kernel():
    mesh = plsc.VectorSubcoreMesh(core_axis_name="c", subcore_axis_name="s",
                                  num_cores=2)

    @functools.partial(
        pl.kernel, mesh=mesh,
        out_type=jax.ShapeDtypeStruct((NPAD, D), jnp.float32),
        scratch_types=[
            pltpu.VMEM((TPW, D), jnp.float32),
            pltpu.VMEM((TPW,), jnp.int32),
            pltpu.VMEM((TPW,), jnp.int32),
            pltpu.SemaphoreType.DMA,
            pltpu.SemaphoreType.DMA,
        ])
    def disp(hn2_hbm, p0_hbm, p1_hbm, x_hbm, rows_v, p0_v, p1_v, s0, s1):
        wid = jax.lax.axis_index("s") * 2 + jax.lax.axis_index("c")
        base = wid * TPW
        pltpu.sync_copy(hn2_hbm.at[pl.ds(base, TPW)], rows_v)
        pltpu.sync_copy(p0_hbm.at[pl.ds(base, TPW)], p0_v)
        pltpu.sync_copy(p1_hbm.at[pl.ds(base, TPW)], p1_v)
        c0 = pltpu.async_copy(rows_v, x_hbm.at[p0_v], s0)
        c1 = pltpu.async_copy(rows_v, x_hbm.at[p1_v], s1)
        c0.wait()
        c1.wait()

    return disp


def _sc_dispatch(hn2, p0f, p1f):
    return _sc_dispatch_kernel()(hn2, p0f, p1f)


@functools.cache
def _sc_combine_kernel():
    mesh = plsc.VectorSubcoreMesh(core_axis_name="c", subcore_axis_name="s",
                                  num_cores=2)

    @functools.partial(
        pl.kernel, mesh=mesh,
        out_type=(jax.ShapeDtypeStruct((T, D), jnp.float32),
                  jax.ShapeDtypeStruct((T, D), jnp.float32)),
        scratch_types=[
            pltpu.VMEM((TPW, D), jnp.float32),
            pltpu.VMEM((TPW,), jnp.int32),
            pltpu.VMEM((TPW,), jnp.int32),
            pltpu.SemaphoreType.DMA,
        ])
    def comb(y_hbm, p0_hbm, p1_hbm, yg0_hbm, yg1_hbm,
             rows_v, p0_v, p1_v, sem):
        wid = jax.lax.axis_index("s") * 2 + jax.lax.axis_index("c")
        base = wid * TPW
        pltpu.sync_copy(p0_hbm.at[pl.ds(base, TPW)], p0_v)
        pltpu.sync_copy(p1_hbm.at[pl.ds(base, TPW)], p1_v)
        pltpu.async_copy(y_hbm.at[p0_v], rows_v, sem).wait()
        pltpu.sync_copy(rows_v, yg0_hbm.at[pl.ds(base, TPW)])
        pltpu.async_copy(y_hbm.at[p1_v], rows_v, sem).wait()
        pltpu.sync_copy(rows_v, yg1_hbm.at[pl.ds(base, TPW)])

    return comb


def _sc_combine(y, p0f, p1f):
    return _sc_combine_kernel()(y, p0f, p1f)


def _k4_body(te_ref, x_ref, wg_ref, wu_ref, wd_ref, y_ref):
    x = x_ref[...].astype(jnp.bfloat16)
    wg = wg_ref[0].astype(jnp.bfloat16)
    wu = wu_ref[0].astype(jnp.bfloat16)
    wd = wd_ref[0].astype(jnp.bfloat16)
    g = jax.lax.dot_general(x, wg, (((1,), (1,)), ((), ())),
                            preferred_element_type=jnp.float32)
    u = jax.lax.dot_general(x, wu, (((1,), (1,)), ((), ())),
                            preferred_element_type=jnp.float32)
    a = (g * jax.nn.sigmoid(g) * u).astype(jnp.bfloat16)
    y_ref[...] = jax.lax.dot_general(a, wd, (((1,), (1,)), ((), ())),
                                     preferred_element_type=jnp.float32)


def _k4(te, x_sorted, w_gate, w_up, w_down):
    grid_spec = pltpu.PrefetchScalarGridSpec(
        num_scalar_prefetch=1,
        grid=(NRT,),
        in_specs=[
            pl.BlockSpec((BTM, D), lambda i, te_r: (i, 0)),
            pl.BlockSpec((1, DFF, D), lambda i, te_r: (te_r[i], 0, 0)),
            pl.BlockSpec((1, DFF, D), lambda i, te_r: (te_r[i], 0, 0)),
            pl.BlockSpec((1, D, DFF), lambda i, te_r: (te_r[i], 0, 0)),
        ],
        out_specs=pl.BlockSpec((BTM, D), lambda i, te_r: (i, 0)),
    )
    return pl.pallas_call(
        _k4_body,
        grid_spec=grid_spec,
        out_shape=jax.ShapeDtypeStruct((NPAD, D), jnp.float32),
    )(te, x_sorted, w_gate, w_up, w_down)


def _k5a_body(h2_ref, hn2_ref, sg_ref, su_ref, sd_ref, out_ref):
    x = hn2_ref[...].astype(jnp.bfloat16)
    g = jax.lax.dot_general(x, sg_ref[...], (((1,), (1,)), ((), ())),
                            preferred_element_type=jnp.float32)
    u = jax.lax.dot_general(x, su_ref[...], (((1,), (1,)), ((), ())),
                            preferred_element_type=jnp.float32)
    a = (g * jax.nn.sigmoid(g) * u).astype(jnp.bfloat16)
    sh = jax.lax.dot_general(a, sd_ref[...], (((1,), (1,)), ((), ())),
                             preferred_element_type=jnp.float32)
    out_ref[...] = h2_ref[...] + sh


def _k5a(h2, hn2, sg, su, sd):
    return pl.pallas_call(
        _k5a_body,
        grid=(NBT,),
        in_specs=[
            pl.BlockSpec((BT, D), lambda i: (i, 0)),
            pl.BlockSpec((BT, D), lambda i: (i, 0)),
            pl.BlockSpec((DFF, D), lambda i: (0, 0)),
            pl.BlockSpec((DFF, D), lambda i: (0, 0)),
            pl.BlockSpec((D, DFF), lambda i: (0, 0)),
        ],
        out_specs=pl.BlockSpec((BT, D), lambda i: (i, 0)),
        out_shape=jax.ShapeDtypeStruct((T, D), jnp.float32),
    )(h2, hn2, sg, su, sd)


def _k5b_body(base_ref, yg0_ref, yg1_ref, w0_ref, w1_ref, out_ref):
    out_ref[...] = (base_ref[...]
                    + yg0_ref[...] * w0_ref[...]
                    + yg1_ref[...] * w1_ref[...])


def _k5b(base, yg0, yg1, w0, w1):
    return pl.pallas_call(
        _k5b_body,
        grid=(NBT,),
        in_specs=[
            pl.BlockSpec((BT, D), lambda i: (i, 0)),
            pl.BlockSpec((BT, D), lambda i: (i, 0)),
            pl.BlockSpec((BT, D), lambda i: (i, 0)),
            pl.BlockSpec((BT, 1), lambda i: (i, 0)),
            pl.BlockSpec((BT, 1), lambda i: (i, 0)),
        ],
        out_specs=pl.BlockSpec((BT, D), lambda i: (i, 0)),
        out_shape=jax.ShapeDtypeStruct((T, D), jnp.float32),
    )(base, yg0, yg1, w0, w1)


@jax.jit
def _run(hidden_states, qkv_w, attn_gate_w, o_w, q_norm_w, k_norm_w,
         ln1_w, ln2_w, router_w, expert_bias, w_gate, w_up, w_down,
         sh_gate, sh_up, sh_down):
    h = hidden_states
    q, k, v, gate = _k1(h, ln1_w, qkv_w, attn_gate_w, q_norm_w, k_norm_w)

    o = _k2(q, k, v)
    o2 = o.transpose(1, 0, 2).reshape(T, NH * HD)

    h2, hn2, cmb, rank, off = _k3(o2, gate, h, o_w, ln2_w, router_w,
                                  expert_bias)
    p0, p1, w0, w1, te = _k3b(rank, cmb, off)
    p0f = p0.reshape(T)
    p1f = p1.reshape(T)

    x_sorted = _sc_dispatch(hn2, p0f, p1f)
    y = _k4(te.reshape(NRT), x_sorted, w_gate, w_up, w_down)
    yg0, yg1 = _sc_combine(y, p0f, p1f)

    sg = sh_gate.astype(jnp.bfloat16)
    su = sh_up.astype(jnp.bfloat16)
    sd = sh_down.astype(jnp.bfloat16)
    base = _k5a(h2, hn2, sg, su, sd)
    return _k5b(base, yg0, yg1, w0, w1)


def kernel(positions, hidden_states, qkv_w, attn_gate_w, o_w, q_norm_w,
           k_norm_w, ln1_w, ln2_w, router_w, expert_bias, w_gate, w_up,
           w_down, sh_gate, sh_up, sh_down):
    return _run(hidden_states, qkv_w, attn_gate_w, o_w, q_norm_w, k_norm_w,
                ln1_w, ln2_w, router_w, expert_bias, w_gate, w_up, w_down,
                sh_gate, sh_up, sh_down)


# speedup vs baseline: 1.0447x; 1.0447x over previous
import functools
import jax
import jax.numpy as jnp
from jax.experimental import pallas as pl
from jax.experimental.pallas import tpu as pltpu
from jax.experimental.pallas import tpu_sc as plsc

T = 2048
D = 1024
NH = 16
NKV = 4
HD = 64
E = 8
TOPK = 2
NG = 4
TG = 2
DFF = 512
EPS = 1e-05

BT = 256
NBT = T // BT
BQ = 512
NBQ = T // BQ

BTM = 256
NRT = 24
NPAD = NRT * BTM
NWORK = 32
TPW = T // NWORK

_HIGH = jax.lax.Precision.HIGHEST


def _rms(x, w, eps=EPS):
    v = jnp.mean(x * x, axis=-1, keepdims=True)
    return x * jax.lax.rsqrt(v + eps) * w


def _dot_nt(a, b):
    return jax.lax.dot_general(
        a.astype(jnp.bfloat16), b.astype(jnp.bfloat16),
        (((1,), (1,)), ((), ())), preferred_element_type=jnp.float32)


def _k1_body(h_ref, ln1_ref, qkvw_ref, gatew_ref, qn_ref, kn_ref,
             q_out, k_out, v_out, gate_out):
    x = h_ref[...]
    xn = _rms(x, ln1_ref[...])
    y = _dot_nt(xn, qkvw_ref[...])
    gate_out[...] = _dot_nt(xn, gatew_ref[...])
    qn = qn_ref[...]
    kn = kn_ref[...]
    for h in range(NH):
        qh = _rms(y[:, h * HD:(h + 1) * HD], qn) * (HD ** -0.5)
        q_out[h] = qh.astype(jnp.bfloat16)
    for h in range(NKV):
        kb = NH * HD + h * HD
        k_out[h] = _rms(y[:, kb:kb + HD], kn).astype(jnp.bfloat16)
        vb = (NH + NKV) * HD + h * HD
        v_out[h] = y[:, vb:vb + HD].astype(jnp.bfloat16)


def _k1(h, ln1_w, qkv_w, attn_gate_w, q_norm_w, k_norm_w):
    return pl.pallas_call(
        _k1_body,
        grid=(NBT,),
        in_specs=[
            pl.BlockSpec((BT, D), lambda i: (i, 0)),
            pl.BlockSpec((1, D), lambda i: (0, 0)),
            pl.BlockSpec(((NH + 2 * NKV) * HD, D), lambda i: (0, 0)),
            pl.BlockSpec((NH * HD, D), lambda i: (0, 0)),
            pl.BlockSpec((1, HD), lambda i: (0, 0)),
            pl.BlockSpec((1, HD), lambda i: (0, 0)),
        ],
        out_specs=[
            pl.BlockSpec((NH, BT, HD), lambda i: (0, i, 0)),
            pl.BlockSpec((NKV, BT, HD), lambda i: (0, i, 0)),
            pl.BlockSpec((NKV, BT, HD), lambda i: (0, i, 0)),
            pl.BlockSpec((BT, NH * HD), lambda i: (i, 0)),
        ],
        out_shape=[
            jax.ShapeDtypeStruct((NH, T, HD), jnp.bfloat16),
            jax.ShapeDtypeStruct((NKV, T, HD), jnp.bfloat16),
            jax.ShapeDtypeStruct((NKV, T, HD), jnp.bfloat16),
            jax.ShapeDtypeStruct((T, NH * HD), jnp.float32),
        ],
    )(h, ln1_w.reshape(1, D), qkv_w, attn_gate_w,
      q_norm_w.reshape(1, HD), k_norm_w.reshape(1, HD))


def _attn_body(q_ref, k_ref, v_ref, o_ref):
    i = pl.program_id(1)
    qb = q_ref[0]

    def pexp(s):
        return jnp.exp(jnp.minimum(s, 30.0))

    def body(j, carry):
        l, acc = carry
        kb = k_ref[0, pl.ds(j * BQ, BQ), :]
        vb = v_ref[0, pl.ds(j * BQ, BQ), :]
        s = jax.lax.dot_general(qb, kb, (((1,), (1,)), ((), ())),
                                preferred_element_type=jnp.float32)
        p = pexp(s)
        l = l + jnp.sum(p, axis=-1, keepdims=True)
        acc = acc + jax.lax.dot_general(
            p.astype(jnp.bfloat16), vb, (((1,), (0,)), ((), ())),
            preferred_element_type=jnp.float32)
        return l, acc

    l0 = jnp.zeros((BQ, 1), jnp.float32)
    a0 = jnp.zeros((BQ, HD), jnp.float32)
    l, acc = jax.lax.fori_loop(0, i, body, (l0, a0))

    rows = jax.lax.broadcasted_iota(jnp.int32, (BQ, BQ), 0)
    cols = jax.lax.broadcasted_iota(jnp.int32, (BQ, BQ), 1)
    maskf = (cols <= rows).astype(jnp.float32)
    kb = k_ref[0, pl.ds(i * BQ, BQ), :]
    vb = v_ref[0, pl.ds(i * BQ, BQ), :]
    s = jax.lax.dot_general(qb, kb, (((1,), (1,)), ((), ())),
                            preferred_element_type=jnp.float32)
    p = pexp(s) * maskf
    l = l + jnp.sum(p, axis=-1, keepdims=True)
    acc = acc + jax.lax.dot_general(
        p.astype(jnp.bfloat16), vb, (((1,), (0,)), ((), ())),
        preferred_element_type=jnp.float32)

    o_ref[0] = acc / l


def _k2(q, k, v):
    rep = NH // NKV
    return pl.pallas_call(
        _attn_body,
        grid=(NH, NBQ),
        in_specs=[
            pl.BlockSpec((1, BQ, HD), lambda h, i: (h, i, 0)),
            pl.BlockSpec((1, T, HD), lambda h, i: (h // rep, 0, 0)),
            pl.BlockSpec((1, T, HD), lambda h, i: (h // rep, 0, 0)),
        ],
        out_specs=pl.BlockSpec((1, BQ, HD), lambda h, i: (h, i, 0)),
        out_shape=jax.ShapeDtypeStruct((NH, T, HD), jnp.float32),
    )(q, k, v)


def _rank_lt_t(m, kmax, n):
    rank = jnp.zeros_like(m)
    e_idx = jax.lax.broadcasted_iota(jnp.int32, m.shape, 0)
    for r in range(1, n):
        m_rot = jnp.concatenate([m[r:, :], m[:r, :]], axis=0)
        beat = (m_rot > m) | ((m_rot == m) & (e_idx >= n - r))
        rank = rank + beat.astype(jnp.float32)
    return (rank < kmax).astype(jnp.float32)


def _dot_high(a, b):
    return jax.lax.dot_general(a, b, (((1,), (0,)), ((), ())),
                               preferred_element_type=jnp.float32,
                               precision=_HIGH)


def _lt8():
    ea = jax.lax.broadcasted_iota(jnp.int32, (E, E), 0)
    eb = jax.lax.broadcasted_iota(jnp.int32, (E, E), 1)
    return (eb < ea).astype(jnp.float32)


def _k3_body(o_ref, gate_ref, res_ref, ow_ref, ln2_ref, rw_ref, bias_ref,
             h2_out, hn2_out, cmb_out, rank_out, off_out, cnt_ref):
    i = pl.program_id(0)

    @pl.when(i == 0)
    def _init():
        cnt_ref[...] = jnp.zeros((E, 1), jnp.float32)

    og = o_ref[...] * jax.nn.sigmoid(gate_ref[...])
    h2 = _dot_nt(og, ow_ref[...]) + res_ref[...]
    h2_out[...] = h2
    hn2 = _rms(h2, ln2_ref[...])
    hn2_out[...] = hn2

    logits = jax.lax.dot_general(
        rw_ref[...], hn2, (((1,), (1,)), ((), ())),
        preferred_element_type=jnp.float32, precision=_HIGH)
    scores = jax.nn.sigmoid(logits)
    sfc = scores + bias_ref[...]

    epg = E // NG
    pg = jax.lax.broadcasted_iota(jnp.int32, (NG, E), 0)
    pe = jax.lax.broadcasted_iota(jnp.int32, (NG, E), 1)
    pair = ((pe // epg) == pg).astype(jnp.float32)
    gs = _dot_high(pair, sfc)

    sel_g = _rank_lt_t(gs, TG, NG)
    ee = jax.lax.broadcasted_iota(jnp.int32, (E, NG), 0)
    gg = jax.lax.broadcasted_iota(jnp.int32, (E, NG), 1)
    expand = (gg == (ee // epg)).astype(jnp.float32)
    mask_e = _dot_high(expand, sel_g)

    masked = jnp.where(mask_e > 0.5, sfc, -1e30)
    sel_e = _rank_lt_t(masked, TOPK, E)
    w = scores * sel_e
    denom = jnp.sum(w, axis=0, keepdims=True) + 1e-20
    cmb_out[...] = w / denom

    lr = jax.lax.broadcasted_iota(jnp.int32, (BT, BT), 0)
    lc = jax.lax.broadcasted_iota(jnp.int32, (BT, BT), 1)
    ltl = (lr < lc).astype(jnp.float32)
    rank_blk = _dot_high(sel_e, ltl)
    rank_out[...] = rank_blk + cnt_ref[...]
    new_cnt = cnt_ref[...] + jnp.sum(sel_e, axis=1, keepdims=True)
    cnt_ref[...] = new_cnt
    padded = jnp.floor((new_cnt + (BTM - 1)) * (1.0 / BTM)) * BTM
    off_out[...] = _dot_high(_lt8(), padded)


def _k3(o2, gate, res, o_w, ln2_w, router_w, expert_bias):
    return pl.pallas_call(
        _k3_body,
        grid=(NBT,),
        in_specs=[
            pl.BlockSpec((BT, D), lambda i: (i, 0)),
            pl.BlockSpec((BT, D), lambda i: (i, 0)),
            pl.BlockSpec((BT, D), lambda i: (i, 0)),
            pl.BlockSpec((D, D), lambda i: (0, 0)),
            pl.BlockSpec((1, D), lambda i: (0, 0)),
            pl.BlockSpec((E, D), lambda i: (0, 0)),
            pl.BlockSpec((E, 1), lambda i: (0, 0)),
        ],
        out_specs=[
            pl.BlockSpec((BT, D), lambda i: (i, 0)),
            pl.BlockSpec((BT, D), lambda i: (i, 0)),
            pl.BlockSpec((E, BT), lambda i: (0, i)),
            pl.BlockSpec((E, BT), lambda i: (0, i)),
            pl.BlockSpec((E, 1), lambda i: (0, 0)),
        ],
        out_shape=[
            jax.ShapeDtypeStruct((T, D), jnp.float32),
            jax.ShapeDtypeStruct((T, D), jnp.float32),
            jax.ShapeDtypeStruct((E, T), jnp.float32),
            jax.ShapeDtypeStruct((E, T), jnp.float32),
            jax.ShapeDtypeStruct((E, 1), jnp.float32),
        ],
        scratch_shapes=[pltpu.VMEM((E, 1), jnp.float32)],
    )(o2, gate, res, o_w, ln2_w.reshape(1, D), router_w,
      expert_bias.reshape(E, 1))


def _k3b_body(rank_ref, cmb_ref, off_ref, p0_ref, p1_ref, w0_ref, w1_ref,
              te_ref):
    cmb = cmb_ref[...]
    sel = (cmb > 0).astype(jnp.float32)
    s_excl = _dot_high(_lt8(), sel)
    off = off_ref[...]
    pos = off + rank_ref[...]
    is0 = sel * (s_excl == 0).astype(jnp.float32)
    is1 = sel * (s_excl == 1).astype(jnp.float32)
    p0_ref[...] = jnp.sum(pos * is0, axis=0, keepdims=True).astype(jnp.int32)
    p1_ref[...] = jnp.sum(pos * is1, axis=0, keepdims=True).astype(jnp.int32)
    w0_ref[...] = jnp.sum(cmb * is0, axis=0, keepdims=True)
    w1_ref[...] = jnp.sum(cmb * is1, axis=0, keepdims=True)

    jt = jax.lax.broadcasted_iota(jnp.int32, (1, NRT), 1) * BTM
    te = jnp.zeros((1, NRT), jnp.int32)
    for e in range(1, E):
        te = te + (jt >= off[e:e + 1, :].astype(jnp.int32)).astype(jnp.int32)
    te_ref[...] = te


def _k3b(rank, cmb, off):
    return pl.pallas_call(
        _k3b_body,
        grid=(NBT,),
        in_specs=[
            pl.BlockSpec((E, BT), lambda i: (0, i)),
            pl.BlockSpec((E, BT), lambda i: (0, i)),
            pl.BlockSpec((E, 1), lambda i: (0, 0)),
        ],
        out_specs=[
            pl.BlockSpec((1, BT), lambda i: (0, i)),
            pl.BlockSpec((1, BT), lambda i: (0, i)),
            pl.BlockSpec((1, BT), lambda i: (0, i)),
            pl.BlockSpec((1, BT), lambda i: (0, i)),
            pl.BlockSpec((1, NRT), lambda i: (0, 0)),
        ],
        out_shape=[
            jax.ShapeDtypeStruct((1, T), jnp.int32),
            jax.ShapeDtypeStruct((1, T), jnp.int32),
            jax.ShapeDtypeStruct((1, T), jnp.float32),
            jax.ShapeDtypeStruct((1, T), jnp.float32),
            jax.ShapeDtypeStruct((1, NRT), jnp.int32),
        ],
    )(rank, cmb, off)


@functools.cache
def _sc_dispatch_kernel():
    mesh = plsc.VectorSubcoreMesh(core_axis_name="c", subcore_axis_name="s",
                                  num_cores=2)

    @functools.partial(
        pl.kernel, mesh=mesh,
        out_type=jax.ShapeDtypeStruct((NPAD, D), jnp.float32),
        scratch_types=[
            pltpu.VMEM((TPW, D), jnp.float32),
            pltpu.VMEM((TPW,), jnp.int32),
            pltpu.VMEM((TPW,), jnp.int32),
            pltpu.SemaphoreType.DMA,
            pltpu.SemaphoreType.DMA,
        ])
    def disp(hn2_hbm, p0_hbm, p1_hbm, x_hbm, rows_v, p0_v, p1_v, s0, s1):
        wid = jax.lax.axis_index("s") * 2 + jax.lax.axis_index("c")
        base = wid * TPW
        pltpu.sync_copy(hn2_hbm.at[pl.ds(base, TPW)], rows_v)
        pltpu.sync_copy(p0_hbm.at[pl.ds(base, TPW)], p0_v)
        pltpu.sync_copy(p1_hbm.at[pl.ds(base, TPW)], p1_v)
        c0 = pltpu.async_copy(rows_v, x_hbm.at[p0_v], s0)
        c1 = pltpu.async_copy(rows_v, x_hbm.at[p1_v], s1)
        c0.wait()
        c1.wait()

    return disp


def _sc_dispatch(hn2, p0f, p1f):
    return _sc_dispatch_kernel()(hn2, p0f, p1f)


@functools.cache
def _sc_combine_kernel():
    mesh = plsc.VectorSubcoreMesh(core_axis_name="c", subcore_axis_name="s",
                                  num_cores=2)

    @functools.partial(
        pl.kernel, mesh=mesh,
        out_type=(jax.ShapeDtypeStruct((T, D), jnp.float32),
                  jax.ShapeDtypeStruct((T, D), jnp.float32)),
        scratch_types=[
            pltpu.VMEM((TPW, D), jnp.float32),
            pltpu.VMEM((TPW,), jnp.int32),
            pltpu.VMEM((TPW,), jnp.int32),
            pltpu.SemaphoreType.DMA,
        ])
    def comb(y_hbm, p0_hbm, p1_hbm, yg0_hbm, yg1_hbm,
             rows_v, p0_v, p1_v, sem):
        wid = jax.lax.axis_index("s") * 2 + jax.lax.axis_index("c")
        base = wid * TPW
        pltpu.sync_copy(p0_hbm.at[pl.ds(base, TPW)], p0_v)
        pltpu.sync_copy(p1_hbm.at[pl.ds(base, TPW)], p1_v)
        pltpu.async_copy(y_hbm.at[p0_v], rows_v, sem).wait()
        pltpu.sync_copy(rows_v, yg0_hbm.at[pl.ds(base, TPW)])
        pltpu.async_copy(y_hbm.at[p1_v], rows_v, sem).wait()
        pltpu.sync_copy(rows_v, yg1_hbm.at[pl.ds(base, TPW)])

    return comb


def _sc_combine(y, p0f, p1f):
    return _sc_combine_kernel()(y, p0f, p1f)


def _k4_body(te_ref, x_ref, wg_ref, wu_ref, wd_ref, y_ref):
    x = x_ref[...].astype(jnp.bfloat16)
    wg = wg_ref[0].astype(jnp.bfloat16)
    wu = wu_ref[0].astype(jnp.bfloat16)
    wd = wd_ref[0].astype(jnp.bfloat16)
    g = jax.lax.dot_general(x, wg, (((1,), (1,)), ((), ())),
                            preferred_element_type=jnp.float32)
    u = jax.lax.dot_general(x, wu, (((1,), (1,)), ((), ())),
                            preferred_element_type=jnp.float32)
    a = (g * jax.nn.sigmoid(g) * u).astype(jnp.bfloat16)
    y_ref[...] = jax.lax.dot_general(a, wd, (((1,), (1,)), ((), ())),
                                     preferred_element_type=jnp.float32)


def _k4(te, x_sorted, w_gate, w_up, w_down):
    grid_spec = pltpu.PrefetchScalarGridSpec(
        num_scalar_prefetch=1,
        grid=(NRT,),
        in_specs=[
            pl.BlockSpec((BTM, D), lambda i, te_r: (i, 0)),
            pl.BlockSpec((1, DFF, D), lambda i, te_r: (te_r[i], 0, 0)),
            pl.BlockSpec((1, DFF, D), lambda i, te_r: (te_r[i], 0, 0)),
            pl.BlockSpec((1, D, DFF), lambda i, te_r: (te_r[i], 0, 0)),
        ],
        out_specs=pl.BlockSpec((BTM, D), lambda i, te_r: (i, 0)),
    )
    return pl.pallas_call(
        _k4_body,
        grid_spec=grid_spec,
        out_shape=jax.ShapeDtypeStruct((NPAD, D), jnp.float32),
    )(te, x_sorted, w_gate, w_up, w_down)


def _k5a_body(h2_ref, hn2_ref, sg_ref, su_ref, sd_ref, out_ref):
    x = hn2_ref[...].astype(jnp.bfloat16)
    g = jax.lax.dot_general(x, sg_ref[...], (((1,), (1,)), ((), ())),
                            preferred_element_type=jnp.float32)
    u = jax.lax.dot_general(x, su_ref[...], (((1,), (1,)), ((), ())),
                            preferred_element_type=jnp.float32)
    a = (g * jax.nn.sigmoid(g) * u).astype(jnp.bfloat16)
    sh = jax.lax.dot_general(a, sd_ref[...], (((1,), (1,)), ((), ())),
                             preferred_element_type=jnp.float32)
    out_ref[...] = h2_ref[...] + sh


def _k5a(h2, hn2, sg, su, sd):
    return pl.pallas_call(
        _k5a_body,
        grid=(NBT,),
        in_specs=[
            pl.BlockSpec((BT, D), lambda i: (i, 0)),
            pl.BlockSpec((BT, D), lambda i: (i, 0)),
            pl.BlockSpec((DFF, D), lambda i: (0, 0)),
            pl.BlockSpec((DFF, D), lambda i: (0, 0)),
            pl.BlockSpec((D, DFF), lambda i: (0, 0)),
        ],
        out_specs=pl.BlockSpec((BT, D), lambda i: (i, 0)),
        out_shape=jax.ShapeDtypeStruct((T, D), jnp.float32),
    )(h2, hn2, sg, su, sd)


def _k5b_body(base_ref, yg0_ref, yg1_ref, w0_ref, w1_ref, out_ref):
    out_ref[...] = (base_ref[...]
                    + yg0_ref[...] * w0_ref[...]
                    + yg1_ref[...] * w1_ref[...])


def _k5b(base, yg0, yg1, w0, w1):
    return pl.pallas_call(
        _k5b_body,
        grid=(NBT,),
        in_specs=[
            pl.BlockSpec((BT, D), lambda i: (i, 0)),
            pl.BlockSpec((BT, D), lambda i: (i, 0)),
            pl.BlockSpec((BT, D), lambda i: (i, 0)),
            pl.BlockSpec((BT, 1), lambda i: (i, 0)),
            pl.BlockSpec((BT, 1), lambda i: (i, 0)),
        ],
        out_specs=pl.BlockSpec((BT, D), lambda i: (i, 0)),
        out_shape=jax.ShapeDtypeStruct((T, D), jnp.float32),
    )(base, yg0, yg1, w0, w1)


@jax.jit
def _run(hidden_states, qkv_w, attn_gate_w, o_w, q_norm_w, k_norm_w,
         ln1_w, ln2_w, router_w, expert_bias, w_gate, w_up, w_down,
         sh_gate, sh_up, sh_down):
    h = hidden_states
    q, k, v, gate = _k1(h, ln1_w, qkv_w, attn_gate_w, q_norm_w, k_norm_w)

    o = _k2(q, k, v)
    o2 = o.transpose(1, 0, 2).reshape(T, NH * HD)

    h2, hn2, cmb, rank, off = _k3(o2, gate, h, o_w, ln2_w, router_w,
                                  expert_bias)
    p0, p1, w0, w1, te = _k3b(rank, cmb, off)
    p0f = p0.reshape(T)
    p1f = p1.reshape(T)

    x_sorted = _sc_dispatch(hn2, p0f, p1f)
    y = _k4(te.reshape(NRT), x_sorted, w_gate, w_up, w_down)
    yg0, yg1 = _sc_combine(y, p0f, p1f)

    sg = sh_gate.astype(jnp.bfloat16)
    su = sh_up.astype(jnp.bfloat16)
    sd = sh_down.astype(jnp.bfloat16)
    base = _k5a(h2, hn2, sg, su, sd)
    return _k5b(base, yg0, yg1, w0.reshape(T, 1), w1.reshape(T, 1))


def kernel(positions, hidden_states, qkv_w, attn_gate_w, o_w, q_norm_w,
           k_norm_w, ln1_w, ln2_w, router_w, expert_bias, w_gate, w_up,
           w_down, sh_gate, sh_up, sh_down):
    return _run(hidden_states, qkv_w, attn_gate_w, o_w, q_norm_w, k_norm_w,
                ln1_w, ln2_w, router_w, expert_bias, w_gate, w_up, w_down,
                sh_gate, sh_up, sh_down)


# speedup vs baseline: 1.1283x; 1.0800x over previous
import functools
import jax
import jax.numpy as jnp
from jax.experimental import pallas as pl
from jax.experimental.pallas import tpu as pltpu
from jax.experimental.pallas import tpu_sc as plsc

T = 2048
D = 1024
NH = 16
NKV = 4
HD = 64
E = 8
TOPK = 2
NG = 4
TG = 2
DFF = 512
EPS = 1e-05

BT = 256
NBT = T // BT
BQ = 512
NBQ = T // BQ

BTM = 256
NRT = 24
NPAD = NRT * BTM
NWORK = 32
TPW = T // NWORK

_HIGH = jax.lax.Precision.HIGHEST


def _rms(x, w, eps=EPS):
    v = jnp.mean(x * x, axis=-1, keepdims=True)
    return x * jax.lax.rsqrt(v + eps) * w


def _dot_nt(a, b):
    return jax.lax.dot_general(
        a.astype(jnp.bfloat16), b.astype(jnp.bfloat16),
        (((1,), (1,)), ((), ())), preferred_element_type=jnp.float32)


def _k1_body(h_ref, ln1_ref, qkvw_ref, gatew_ref, qn_ref, kn_ref,
             q_out, k_out, v_out, gate_out):
    x = h_ref[...]
    xn = _rms(x, ln1_ref[...])
    y = _dot_nt(xn, qkvw_ref[...])
    gate_out[...] = _dot_nt(xn, gatew_ref[...]).astype(jnp.bfloat16)
    qn = qn_ref[...]
    kn = kn_ref[...]
    for h in range(NH):
        qh = _rms(y[:, h * HD:(h + 1) * HD], qn) * (HD ** -0.5)
        q_out[h] = qh.astype(jnp.bfloat16)
    for h in range(NKV):
        kb = NH * HD + h * HD
        k_out[h] = _rms(y[:, kb:kb + HD], kn).astype(jnp.bfloat16)
        vb = (NH + NKV) * HD + h * HD
        v_out[h] = y[:, vb:vb + HD].astype(jnp.bfloat16)


def _k1(h, ln1_w, qkv_w, attn_gate_w, q_norm_w, k_norm_w):
    return pl.pallas_call(
        _k1_body,
        grid=(NBT,),
        in_specs=[
            pl.BlockSpec((BT, D), lambda i: (i, 0)),
            pl.BlockSpec((1, D), lambda i: (0, 0)),
            pl.BlockSpec(((NH + 2 * NKV) * HD, D), lambda i: (0, 0)),
            pl.BlockSpec((NH * HD, D), lambda i: (0, 0)),
            pl.BlockSpec((1, HD), lambda i: (0, 0)),
            pl.BlockSpec((1, HD), lambda i: (0, 0)),
        ],
        out_specs=[
            pl.BlockSpec((NH, BT, HD), lambda i: (0, i, 0)),
            pl.BlockSpec((NKV, BT, HD), lambda i: (0, i, 0)),
            pl.BlockSpec((NKV, BT, HD), lambda i: (0, i, 0)),
            pl.BlockSpec((BT, NH * HD), lambda i: (i, 0)),
        ],
        out_shape=[
            jax.ShapeDtypeStruct((NH, T, HD), jnp.bfloat16),
            jax.ShapeDtypeStruct((NKV, T, HD), jnp.bfloat16),
            jax.ShapeDtypeStruct((NKV, T, HD), jnp.bfloat16),
            jax.ShapeDtypeStruct((T, NH * HD), jnp.bfloat16),
        ],
    )(h, ln1_w.reshape(1, D), qkv_w, attn_gate_w,
      q_norm_w.reshape(1, HD), k_norm_w.reshape(1, HD))


def _attn_body(q_ref, k_ref, v_ref, mask_ref, o_ref):
    i = pl.program_id(1)
    qa = q_ref[0]
    qb = q_ref[1]

    def pexp(s):
        return jnp.exp(jnp.minimum(s, 30.0))

    def chunk(kb, vb, carry, maskf=None):
        la, aa, lb, ab = carry
        sa = jax.lax.dot_general(qa, kb, (((1,), (1,)), ((), ())),
                                 preferred_element_type=jnp.float32)
        sb = jax.lax.dot_general(qb, kb, (((1,), (1,)), ((), ())),
                                 preferred_element_type=jnp.float32)
        pa = pexp(sa)
        pb = pexp(sb)
        if maskf is not None:
            pa = pa * maskf
            pb = pb * maskf
        la = la + jnp.sum(pa, axis=-1, keepdims=True)
        lb = lb + jnp.sum(pb, axis=-1, keepdims=True)
        aa = aa + jax.lax.dot_general(
            pa.astype(jnp.bfloat16), vb, (((1,), (0,)), ((), ())),
            preferred_element_type=jnp.float32)
        ab = ab + jax.lax.dot_general(
            pb.astype(jnp.bfloat16), vb, (((1,), (0,)), ((), ())),
            preferred_element_type=jnp.float32)
        return la, aa, lb, ab

    def body(j, carry):
        kb = k_ref[0, pl.ds(j * BQ, BQ), :]
        vb = v_ref[0, pl.ds(j * BQ, BQ), :]
        return chunk(kb, vb, carry)

    z1 = jnp.zeros((BQ, 1), jnp.float32)
    za = jnp.zeros((BQ, HD), jnp.float32)
    carry = jax.lax.fori_loop(0, i, body, (z1, za, z1, za))

    kb = k_ref[0, pl.ds(i * BQ, BQ), :]
    vb = v_ref[0, pl.ds(i * BQ, BQ), :]
    la, aa, lb, ab = chunk(kb, vb, carry, mask_ref[...])

    o_ref[0] = (aa / la).astype(jnp.bfloat16)
    o_ref[1] = (ab / lb).astype(jnp.bfloat16)


def _k2(q, k, v, mask):
    rep = NH // NKV
    return pl.pallas_call(
        _attn_body,
        grid=(NH // 2, NBQ),
        in_specs=[
            pl.BlockSpec((2, BQ, HD), lambda h, i: (h, i, 0)),
            pl.BlockSpec((1, T, HD), lambda h, i: (h // 2, 0, 0)),
            pl.BlockSpec((1, T, HD), lambda h, i: (h // 2, 0, 0)),
            pl.BlockSpec((BQ, BQ), lambda h, i: (0, 0)),
        ],
        out_specs=pl.BlockSpec((2, BQ, HD), lambda h, i: (h, i, 0)),
        out_shape=jax.ShapeDtypeStruct((NH, T, HD), jnp.bfloat16),
    )(q, k, v, mask)


def _rank_lt_t(m, kmax, n):
    rank = jnp.zeros_like(m)
    e_idx = jax.lax.broadcasted_iota(jnp.int32, m.shape, 0)
    for r in range(1, n):
        m_rot = jnp.concatenate([m[r:, :], m[:r, :]], axis=0)
        beat = (m_rot > m) | ((m_rot == m) & (e_idx >= n - r))
        rank = rank + beat.astype(jnp.float32)
    return (rank < kmax).astype(jnp.float32)


def _dot_high(a, b):
    return jax.lax.dot_general(a, b, (((1,), (0,)), ((), ())),
                               preferred_element_type=jnp.float32,
                               precision=_HIGH)


def _lt8():
    ea = jax.lax.broadcasted_iota(jnp.int32, (E, E), 0)
    eb = jax.lax.broadcasted_iota(jnp.int32, (E, E), 1)
    return (eb < ea).astype(jnp.float32)


def _k3_body(o_ref, gate_ref, res_ref, ow_ref, ln2_ref, rw_ref, bias_ref,
             h2_out, hn2_out, cmb_out, rank_out, off_out, cnt_ref):
    i = pl.program_id(0)

    @pl.when(i == 0)
    def _init():
        cnt_ref[...] = jnp.zeros((E, 1), jnp.float32)

    og = o_ref[...] * jax.nn.sigmoid(gate_ref[...])
    h2 = _dot_nt(og, ow_ref[...]) + res_ref[...]
    h2_out[...] = h2
    hn2 = _rms(h2, ln2_ref[...])
    hn2_out[...] = hn2

    logits = jax.lax.dot_general(
        rw_ref[...], hn2, (((1,), (1,)), ((), ())),
        preferred_element_type=jnp.float32, precision=_HIGH)
    scores = jax.nn.sigmoid(logits)
    sfc = scores + bias_ref[...]

    epg = E // NG
    pg = jax.lax.broadcasted_iota(jnp.int32, (NG, E), 0)
    pe = jax.lax.broadcasted_iota(jnp.int32, (NG, E), 1)
    pair = ((pe // epg) == pg).astype(jnp.float32)
    gs = _dot_high(pair, sfc)

    sel_g = _rank_lt_t(gs, TG, NG)
    ee = jax.lax.broadcasted_iota(jnp.int32, (E, NG), 0)
    gg = jax.lax.broadcasted_iota(jnp.int32, (E, NG), 1)
    expand = (gg == (ee // epg)).astype(jnp.float32)
    mask_e = _dot_high(expand, sel_g)

    masked = jnp.where(mask_e > 0.5, sfc, -1e30)
    sel_e = _rank_lt_t(masked, TOPK, E)
    w = scores * sel_e
    denom = jnp.sum(w, axis=0, keepdims=True) + 1e-20
    cmb_out[...] = w / denom

    lr = jax.lax.broadcasted_iota(jnp.int32, (BT, BT), 0)
    lc = jax.lax.broadcasted_iota(jnp.int32, (BT, BT), 1)
    ltl = (lr < lc).astype(jnp.float32)
    rank_blk = _dot_high(sel_e, ltl)
    rank_out[...] = rank_blk + cnt_ref[...]
    new_cnt = cnt_ref[...] + jnp.sum(sel_e, axis=1, keepdims=True)
    cnt_ref[...] = new_cnt
    padded = jnp.floor((new_cnt + (BTM - 1)) * (1.0 / BTM)) * BTM
    off_out[...] = _dot_high(_lt8(), padded)


def _k3(o2, gate, res, o_w, ln2_w, router_w, expert_bias):
    return pl.pallas_call(
        _k3_body,
        grid=(NBT,),
        in_specs=[
            pl.BlockSpec((BT, D), lambda i: (i, 0)),
            pl.BlockSpec((BT, D), lambda i: (i, 0)),
            pl.BlockSpec((BT, D), lambda i: (i, 0)),
            pl.BlockSpec((D, D), lambda i: (0, 0)),
            pl.BlockSpec((1, D), lambda i: (0, 0)),
            pl.BlockSpec((E, D), lambda i: (0, 0)),
            pl.BlockSpec((E, 1), lambda i: (0, 0)),
        ],
        out_specs=[
            pl.BlockSpec((BT, D), lambda i: (i, 0)),
            pl.BlockSpec((BT, D), lambda i: (i, 0)),
            pl.BlockSpec((E, BT), lambda i: (0, i)),
            pl.BlockSpec((E, BT), lambda i: (0, i)),
            pl.BlockSpec((E, 1), lambda i: (0, 0)),
        ],
        out_shape=[
            jax.ShapeDtypeStruct((T, D), jnp.float32),
            jax.ShapeDtypeStruct((T, D), jnp.float32),
            jax.ShapeDtypeStruct((E, T), jnp.float32),
            jax.ShapeDtypeStruct((E, T), jnp.float32),
            jax.ShapeDtypeStruct((E, 1), jnp.float32),
        ],
        scratch_shapes=[pltpu.VMEM((E, 1), jnp.float32)],
    )(o2, gate, res, o_w, ln2_w.reshape(1, D), router_w,
      expert_bias.reshape(E, 1))


def _k3b_body(rank_ref, cmb_ref, off_ref, p0_ref, p1_ref, w0_ref, w1_ref,
              te_ref):
    cmb = cmb_ref[...]
    sel = (cmb > 0).astype(jnp.float32)
    s_excl = _dot_high(_lt8(), sel)
    off = off_ref[...]
    pos = off + rank_ref[...]
    is0 = sel * (s_excl == 0).astype(jnp.float32)
    is1 = sel * (s_excl == 1).astype(jnp.float32)
    p0_ref[...] = jnp.sum(pos * is0, axis=0, keepdims=True).astype(jnp.int32)
    p1_ref[...] = jnp.sum(pos * is1, axis=0, keepdims=True).astype(jnp.int32)
    w0_ref[...] = jnp.sum(cmb * is0, axis=0, keepdims=True)
    w1_ref[...] = jnp.sum(cmb * is1, axis=0, keepdims=True)

    jt = jax.lax.broadcasted_iota(jnp.int32, (1, NRT), 1) * BTM
    te = jnp.zeros((1, NRT), jnp.int32)
    for e in range(1, E):
        te = te + (jt >= off[e:e + 1, :].astype(jnp.int32)).astype(jnp.int32)
    te_ref[...] = te


def _k3b(rank, cmb, off):
    return pl.pallas_call(
        _k3b_body,
        grid=(NBT,),
        in_specs=[
            pl.BlockSpec((E, BT), lambda i: (0, i)),
            pl.BlockSpec((E, BT), lambda i: (0, i)),
            pl.BlockSpec((E, 1), lambda i: (0, 0)),
        ],
        out_specs=[
            pl.BlockSpec((1, BT), lambda i: (0, i)),
            pl.BlockSpec((1, BT), lambda i: (0, i)),
            pl.BlockSpec((1, BT), lambda i: (0, i)),
            pl.BlockSpec((1, BT), lambda i: (0, i)),
            pl.BlockSpec((1, NRT), lambda i: (0, 0)),
        ],
        out_shape=[
            jax.ShapeDtypeStruct((1, T), jnp.int32),
            jax.ShapeDtypeStruct((1, T), jnp.int32),
            jax.ShapeDtypeStruct((1, T), jnp.float32),
            jax.ShapeDtypeStruct((1, T), jnp.float32),
            jax.ShapeDtypeStruct((1, NRT), jnp.int32),
        ],
    )(rank, cmb, off)


@functools.cache
def _sc_dispatch_kernel():
    mesh = plsc.VectorSubcoreMesh(core_axis_name="c", subcore_axis_name="s",
                                  num_cores=2)

    @functools.partial(
        pl.kernel, mesh=mesh,
        out_type=jax.ShapeDtypeStruct((NPAD, D), jnp.float32),
        scratch_types=[
            pltpu.VMEM((TPW, D), jnp.float32),
            pltpu.VMEM((TPW,), jnp.int32),
            pltpu.VMEM((TPW,), jnp.int32),
            pltpu.SemaphoreType.DMA,
            pltpu.SemaphoreType.DMA,
        ])
    def disp(hn2_hbm, p0_hbm, p1_hbm, x_hbm, rows_v, p0_v, p1_v, s0, s1):
        wid = jax.lax.axis_index("s") * 2 + jax.lax.axis_index("c")
        base = wid * TPW
        pltpu.sync_copy(hn2_hbm.at[pl.ds(base, TPW)], rows_v)
        pltpu.sync_copy(p0_hbm.at[pl.ds(base, TPW)], p0_v)
        pltpu.sync_copy(p1_hbm.at[pl.ds(base, TPW)], p1_v)
        c0 = pltpu.async_copy(rows_v, x_hbm.at[p0_v], s0)
        c1 = pltpu.async_copy(rows_v, x_hbm.at[p1_v], s1)
        c0.wait()
        c1.wait()

    return disp


def _sc_dispatch(hn2, p0f, p1f):
    return _sc_dispatch_kernel()(hn2, p0f, p1f)


@functools.cache
def _sc_combine_kernel():
    mesh = plsc.VectorSubcoreMesh(core_axis_name="c", subcore_axis_name="s",
                                  num_cores=2)

    @functools.partial(
        pl.kernel, mesh=mesh,
        out_type=(jax.ShapeDtypeStruct((T, D), jnp.float32),
                  jax.ShapeDtypeStruct((T, D), jnp.float32)),
        scratch_types=[
            pltpu.VMEM((TPW, D), jnp.float32),
            pltpu.VMEM((TPW,), jnp.int32),
            pltpu.VMEM((TPW,), jnp.int32),
            pltpu.SemaphoreType.DMA,
        ])
    def comb(y_hbm, p0_hbm, p1_hbm, yg0_hbm, yg1_hbm,
             rows_v, p0_v, p1_v, sem):
        wid = jax.lax.axis_index("s") * 2 + jax.lax.axis_index("c")
        base = wid * TPW
        pltpu.sync_copy(p0_hbm.at[pl.ds(base, TPW)], p0_v)
        pltpu.sync_copy(p1_hbm.at[pl.ds(base, TPW)], p1_v)
        pltpu.async_copy(y_hbm.at[p0_v], rows_v, sem).wait()
        pltpu.sync_copy(rows_v, yg0_hbm.at[pl.ds(base, TPW)])
        pltpu.async_copy(y_hbm.at[p1_v], rows_v, sem).wait()
        pltpu.sync_copy(rows_v, yg1_hbm.at[pl.ds(base, TPW)])

    return comb


def _sc_combine(y, p0f, p1f):
    return _sc_combine_kernel()(y, p0f, p1f)


def _k4_body(te_ref, x_ref, wg_ref, wu_ref, wd_ref, y_ref):
    x = x_ref[...].astype(jnp.bfloat16)
    wg = wg_ref[0].astype(jnp.bfloat16)
    wu = wu_ref[0].astype(jnp.bfloat16)
    wd = wd_ref[0].astype(jnp.bfloat16)
    g = jax.lax.dot_general(x, wg, (((1,), (1,)), ((), ())),
                            preferred_element_type=jnp.float32)
    u = jax.lax.dot_general(x, wu, (((1,), (1,)), ((), ())),
                            preferred_element_type=jnp.float32)
    a = (g * jax.nn.sigmoid(g) * u).astype(jnp.bfloat16)
    y_ref[...] = jax.lax.dot_general(a, wd, (((1,), (1,)), ((), ())),
                                     preferred_element_type=jnp.float32)


def _k4(te, x_sorted, w_gate, w_up, w_down):
    grid_spec = pltpu.PrefetchScalarGridSpec(
        num_scalar_prefetch=1,
        grid=(NRT,),
        in_specs=[
            pl.BlockSpec((BTM, D), lambda i, te_r: (i, 0)),
            pl.BlockSpec((1, DFF, D), lambda i, te_r: (te_r[i], 0, 0)),
            pl.BlockSpec((1, DFF, D), lambda i, te_r: (te_r[i], 0, 0)),
            pl.BlockSpec((1, D, DFF), lambda i, te_r: (te_r[i], 0, 0)),
        ],
        out_specs=pl.BlockSpec((BTM, D), lambda i, te_r: (i, 0)),
    )
    return pl.pallas_call(
        _k4_body,
        grid_spec=grid_spec,
        out_shape=jax.ShapeDtypeStruct((NPAD, D), jnp.float32),
    )(te, x_sorted, w_gate, w_up, w_down)


def _k5a_body(h2_ref, hn2_ref, sg_ref, su_ref, sd_ref, out_ref):
    x = hn2_ref[...].astype(jnp.bfloat16)
    g = jax.lax.dot_general(x, sg_ref[...], (((1,), (1,)), ((), ())),
                            preferred_element_type=jnp.float32)
    u = jax.lax.dot_general(x, su_ref[...], (((1,), (1,)), ((), ())),
                            preferred_element_type=jnp.float32)
    a = (g * jax.nn.sigmoid(g) * u).astype(jnp.bfloat16)
    sh = jax.lax.dot_general(a, sd_ref[...], (((1,), (1,)), ((), ())),
                             preferred_element_type=jnp.float32)
    out_ref[...] = h2_ref[...] + sh


def _k5a(h2, hn2, sg, su, sd):
    return pl.pallas_call(
        _k5a_body,
        grid=(NBT,),
        in_specs=[
            pl.BlockSpec((BT, D), lambda i: (i, 0)),
            pl.BlockSpec((BT, D), lambda i: (i, 0)),
            pl.BlockSpec((DFF, D), lambda i: (0, 0)),
            pl.BlockSpec((DFF, D), lambda i: (0, 0)),
            pl.BlockSpec((D, DFF), lambda i: (0, 0)),
        ],
        out_specs=pl.BlockSpec((BT, D), lambda i: (i, 0)),
        out_shape=jax.ShapeDtypeStruct((T, D), jnp.float32),
    )(h2, hn2, sg, su, sd)


def _k5b_body(base_ref, yg0_ref, yg1_ref, w0_ref, w1_ref, out_ref):
    out_ref[...] = (base_ref[...]
                    + yg0_ref[...] * w0_ref[...]
                    + yg1_ref[...] * w1_ref[...])


def _k5b(base, yg0, yg1, w0, w1):
    return pl.pallas_call(
        _k5b_body,
        grid=(NBT,),
        in_specs=[
            pl.BlockSpec((BT, D), lambda i: (i, 0)),
            pl.BlockSpec((BT, D), lambda i: (i, 0)),
            pl.BlockSpec((BT, D), lambda i: (i, 0)),
            pl.BlockSpec((BT, 1), lambda i: (i, 0)),
            pl.BlockSpec((BT, 1), lambda i: (i, 0)),
        ],
        out_specs=pl.BlockSpec((BT, D), lambda i: (i, 0)),
        out_shape=jax.ShapeDtypeStruct((T, D), jnp.float32),
    )(base, yg0, yg1, w0, w1)


@jax.jit
def _run(hidden_states, qkv_w, attn_gate_w, o_w, q_norm_w, k_norm_w,
         ln1_w, ln2_w, router_w, expert_bias, w_gate, w_up, w_down,
         sh_gate, sh_up, sh_down):
    h = hidden_states
    q, k, v, gate = _k1(h, ln1_w, qkv_w, attn_gate_w, q_norm_w, k_norm_w)

    mask = jnp.tril(jnp.ones((BQ, BQ), jnp.float32))
    o = _k2(q, k, v, mask)
    o2 = o.transpose(1, 0, 2).reshape(T, NH * HD)

    h2, hn2, cmb, rank, off = _k3(o2, gate, h, o_w, ln2_w, router_w,
                                  expert_bias)
    p0, p1, w0, w1, te = _k3b(rank, cmb, off)
    p0f = p0.reshape(T)
    p1f = p1.reshape(T)

    x_sorted = _sc_dispatch(hn2, p0f, p1f)
    y = _k4(te.reshape(NRT), x_sorted, w_gate, w_up, w_down)
    yg0, yg1 = _sc_combine(y, p0f, p1f)

    sg = sh_gate.astype(jnp.bfloat16)
    su = sh_up.astype(jnp.bfloat16)
    sd = sh_down.astype(jnp.bfloat16)
    base = _k5a(h2, hn2, sg, su, sd)
    return _k5b(base, yg0, yg1, w0.reshape(T, 1), w1.reshape(T, 1))


def kernel(positions, hidden_states, qkv_w, attn_gate_w, o_w, q_norm_w,
           k_norm_w, ln1_w, ln2_w, router_w, expert_bias, w_gate, w_up,
           w_down, sh_gate, sh_up, sh_down):
    return _run(hidden_states, qkv_w, attn_gate_w, o_w, q_norm_w, k_norm_w,
                ln1_w, ln2_w, router_w, expert_bias, w_gate, w_up, w_down,
                sh_gate, sh_up, sh_down)


# speedup vs baseline: 1.1660x; 1.0335x over previous
import functools
import jax
import jax.numpy as jnp
from jax.experimental import pallas as pl
from jax.experimental.pallas import tpu as pltpu
from jax.experimental.pallas import tpu_sc as plsc

T = 2048
D = 1024
NH = 16
NKV = 4
HD = 64
E = 8
TOPK = 2
NG = 4
TG = 2
DFF = 512
EPS = 1e-05

BT = 256
NBT = T // BT
BQ = 512
NBQ = T // BQ

BTM = 256
NRT = 24
NPAD = NRT * BTM
NWORK = 32
TPW = T // NWORK

_HIGH = jax.lax.Precision.HIGHEST


def _rms(x, w, eps=EPS):
    v = jnp.mean(x * x, axis=-1, keepdims=True)
    return x * jax.lax.rsqrt(v + eps) * w


def _dot_nt(a, b):
    return jax.lax.dot_general(
        a.astype(jnp.bfloat16), b.astype(jnp.bfloat16),
        (((1,), (1,)), ((), ())), preferred_element_type=jnp.float32)


def _k1_body(h_ref, ln1_ref, qkvw_ref, gatew_ref, qn_ref, kn_ref,
             q_out, k_out, v_out, gate_out):
    x = h_ref[...]
    xn = _rms(x, ln1_ref[...])
    y = _dot_nt(xn, qkvw_ref[...])
    gate_out[...] = _dot_nt(xn, gatew_ref[...]).astype(jnp.bfloat16)
    qn = qn_ref[...]
    kn = kn_ref[...]
    for h in range(NH):
        qh = _rms(y[:, h * HD:(h + 1) * HD], qn) * (HD ** -0.5)
        q_out[h] = qh.astype(jnp.bfloat16)
    for h in range(NKV):
        kb = NH * HD + h * HD
        k_out[h] = _rms(y[:, kb:kb + HD], kn).astype(jnp.bfloat16)
        vb = (NH + NKV) * HD + h * HD
        v_out[h] = y[:, vb:vb + HD].astype(jnp.bfloat16)


def _k1(h, ln1_w, qkv_w, attn_gate_w, q_norm_w, k_norm_w):
    return pl.pallas_call(
        _k1_body,
        grid=(NBT,),
        in_specs=[
            pl.BlockSpec((BT, D), lambda i: (i, 0)),
            pl.BlockSpec((1, D), lambda i: (0, 0)),
            pl.BlockSpec(((NH + 2 * NKV) * HD, D), lambda i: (0, 0)),
            pl.BlockSpec((NH * HD, D), lambda i: (0, 0)),
            pl.BlockSpec((1, HD), lambda i: (0, 0)),
            pl.BlockSpec((1, HD), lambda i: (0, 0)),
        ],
        out_specs=[
            pl.BlockSpec((NH, BT, HD), lambda i: (0, i, 0)),
            pl.BlockSpec((NKV, BT, HD), lambda i: (0, i, 0)),
            pl.BlockSpec((NKV, BT, HD), lambda i: (0, i, 0)),
            pl.BlockSpec((BT, NH * HD), lambda i: (i, 0)),
        ],
        out_shape=[
            jax.ShapeDtypeStruct((NH, T, HD), jnp.bfloat16),
            jax.ShapeDtypeStruct((NKV, T, HD), jnp.bfloat16),
            jax.ShapeDtypeStruct((NKV, T, HD), jnp.bfloat16),
            jax.ShapeDtypeStruct((T, NH * HD), jnp.bfloat16),
        ],
    )(h, ln1_w.reshape(1, D), qkv_w, attn_gate_w,
      q_norm_w.reshape(1, HD), k_norm_w.reshape(1, HD))


def _attn_body(q_ref, k_ref, v_ref, mask_ref, o_ref):
    i = pl.program_id(1)
    hpg = NH // NKV
    qs = [q_ref[j] for j in range(hpg)]

    def pexp(s):
        return jnp.exp(jnp.minimum(s, 30.0))

    def chunk(kb, vb, carry, maskf=None):
        ls, accs = carry
        ss = [jax.lax.dot_general(qj, kb, (((1,), (1,)), ((), ())),
                                  preferred_element_type=jnp.float32)
              for qj in qs]
        ps = [pexp(s) for s in ss]
        if maskf is not None:
            ps = [p * maskf for p in ps]
        ls = tuple(l + jnp.sum(p, axis=-1, keepdims=True)
                   for l, p in zip(ls, ps))
        accs = tuple(acc + jax.lax.dot_general(
            p.astype(jnp.bfloat16), vb, (((1,), (0,)), ((), ())),
            preferred_element_type=jnp.float32)
            for acc, p in zip(accs, ps))
        return ls, accs

    def body(j, carry):
        kb = k_ref[0, pl.ds(j * BQ, BQ), :]
        vb = v_ref[0, pl.ds(j * BQ, BQ), :]
        return chunk(kb, vb, carry)

    z1 = jnp.zeros((BQ, 1), jnp.float32)
    za = jnp.zeros((BQ, HD), jnp.float32)
    carry = ((z1,) * hpg, (za,) * hpg)
    carry = jax.lax.fori_loop(0, i, body, carry)

    kb = k_ref[0, pl.ds(i * BQ, BQ), :]
    vb = v_ref[0, pl.ds(i * BQ, BQ), :]
    ls, accs = chunk(kb, vb, carry, mask_ref[...])

    for j in range(hpg):
        o_ref[j] = (accs[j] / ls[j]).astype(jnp.bfloat16)


def _k2(q, k, v, mask):
    rep = NH // NKV
    return pl.pallas_call(
        _attn_body,
        grid=(NKV, NBQ),
        in_specs=[
            pl.BlockSpec((rep, BQ, HD), lambda h, i: (h, i, 0)),
            pl.BlockSpec((1, T, HD), lambda h, i: (h, 0, 0)),
            pl.BlockSpec((1, T, HD), lambda h, i: (h, 0, 0)),
            pl.BlockSpec((BQ, BQ), lambda h, i: (0, 0)),
        ],
        out_specs=pl.BlockSpec((rep, BQ, HD), lambda h, i: (h, i, 0)),
        out_shape=jax.ShapeDtypeStruct((NH, T, HD), jnp.bfloat16),
    )(q, k, v, mask)


def _rank_lt_t(m, kmax, n):
    rank = jnp.zeros_like(m)
    e_idx = jax.lax.broadcasted_iota(jnp.int32, m.shape, 0)
    for r in range(1, n):
        m_rot = jnp.concatenate([m[r:, :], m[:r, :]], axis=0)
        beat = (m_rot > m) | ((m_rot == m) & (e_idx >= n - r))
        rank = rank + beat.astype(jnp.float32)
    return (rank < kmax).astype(jnp.float32)


def _dot_high(a, b):
    return jax.lax.dot_general(a, b, (((1,), (0,)), ((), ())),
                               preferred_element_type=jnp.float32,
                               precision=_HIGH)


def _lt8():
    ea = jax.lax.broadcasted_iota(jnp.int32, (E, E), 0)
    eb = jax.lax.broadcasted_iota(jnp.int32, (E, E), 1)
    return (eb < ea).astype(jnp.float32)


def _k3_body(o_ref, gate_ref, res_ref, ow_ref, ln2_ref, rw_ref, bias_ref,
             h2_out, hn2_out, cmb_out, rank_out, off_out, cnt_ref):
    i = pl.program_id(0)

    @pl.when(i == 0)
    def _init():
        cnt_ref[...] = jnp.zeros((E, 1), jnp.float32)

    og = o_ref[...] * jax.nn.sigmoid(gate_ref[...])
    h2 = _dot_nt(og, ow_ref[...]) + res_ref[...]
    h2_out[...] = h2
    hn2 = _rms(h2, ln2_ref[...])
    hn2_out[...] = hn2

    logits = jax.lax.dot_general(
        rw_ref[...], hn2, (((1,), (1,)), ((), ())),
        preferred_element_type=jnp.float32, precision=_HIGH)
    scores = jax.nn.sigmoid(logits)
    sfc = scores + bias_ref[...]

    epg = E // NG
    pg = jax.lax.broadcasted_iota(jnp.int32, (NG, E), 0)
    pe = jax.lax.broadcasted_iota(jnp.int32, (NG, E), 1)
    pair = ((pe // epg) == pg).astype(jnp.float32)
    gs = _dot_high(pair, sfc)

    sel_g = _rank_lt_t(gs, TG, NG)
    ee = jax.lax.broadcasted_iota(jnp.int32, (E, NG), 0)
    gg = jax.lax.broadcasted_iota(jnp.int32, (E, NG), 1)
    expand = (gg == (ee // epg)).astype(jnp.float32)
    mask_e = _dot_high(expand, sel_g)

    masked = jnp.where(mask_e > 0.5, sfc, -1e30)
    sel_e = _rank_lt_t(masked, TOPK, E)
    w = scores * sel_e
    denom = jnp.sum(w, axis=0, keepdims=True) + 1e-20
    cmb_out[...] = w / denom

    lr = jax.lax.broadcasted_iota(jnp.int32, (BT, BT), 0)
    lc = jax.lax.broadcasted_iota(jnp.int32, (BT, BT), 1)
    ltl = (lr < lc).astype(jnp.float32)
    rank_blk = _dot_high(sel_e, ltl)
    rank_out[...] = rank_blk + cnt_ref[...]
    new_cnt = cnt_ref[...] + jnp.sum(sel_e, axis=1, keepdims=True)
    cnt_ref[...] = new_cnt
    padded = jnp.floor((new_cnt + (BTM - 1)) * (1.0 / BTM)) * BTM
    off_out[...] = _dot_high(_lt8(), padded)


def _k3(o2, gate, res, o_w, ln2_w, router_w, expert_bias):
    return pl.pallas_call(
        _k3_body,
        grid=(NBT,),
        in_specs=[
            pl.BlockSpec((BT, D), lambda i: (i, 0)),
            pl.BlockSpec((BT, D), lambda i: (i, 0)),
            pl.BlockSpec((BT, D), lambda i: (i, 0)),
            pl.BlockSpec((D, D), lambda i: (0, 0)),
            pl.BlockSpec((1, D), lambda i: (0, 0)),
            pl.BlockSpec((E, D), lambda i: (0, 0)),
            pl.BlockSpec((E, 1), lambda i: (0, 0)),
        ],
        out_specs=[
            pl.BlockSpec((BT, D), lambda i: (i, 0)),
            pl.BlockSpec((BT, D), lambda i: (i, 0)),
            pl.BlockSpec((E, BT), lambda i: (0, i)),
            pl.BlockSpec((E, BT), lambda i: (0, i)),
            pl.BlockSpec((E, 1), lambda i: (0, 0)),
        ],
        out_shape=[
            jax.ShapeDtypeStruct((T, D), jnp.float32),
            jax.ShapeDtypeStruct((T, D), jnp.float32),
            jax.ShapeDtypeStruct((E, T), jnp.float32),
            jax.ShapeDtypeStruct((E, T), jnp.float32),
            jax.ShapeDtypeStruct((E, 1), jnp.float32),
        ],
        scratch_shapes=[pltpu.VMEM((E, 1), jnp.float32)],
    )(o2, gate, res, o_w, ln2_w.reshape(1, D), router_w,
      expert_bias.reshape(E, 1))


def _k3b_body(rank_ref, cmb_ref, off_ref, p0_ref, p1_ref, w0_ref, w1_ref,
              te_ref):
    cmb = cmb_ref[...]
    sel = (cmb > 0).astype(jnp.float32)
    s_excl = _dot_high(_lt8(), sel)
    off = off_ref[...]
    pos = off + rank_ref[...]
    is0 = sel * (s_excl == 0).astype(jnp.float32)
    is1 = sel * (s_excl == 1).astype(jnp.float32)
    p0_ref[...] = jnp.sum(pos * is0, axis=0, keepdims=True).astype(jnp.int32)
    p1_ref[...] = jnp.sum(pos * is1, axis=0, keepdims=True).astype(jnp.int32)
    w0_ref[...] = jnp.sum(cmb * is0, axis=0, keepdims=True)
    w1_ref[...] = jnp.sum(cmb * is1, axis=0, keepdims=True)

    jt = jax.lax.broadcasted_iota(jnp.int32, (1, NRT), 1) * BTM
    te = jnp.zeros((1, NRT), jnp.int32)
    for e in range(1, E):
        te = te + (jt >= off[e:e + 1, :].astype(jnp.int32)).astype(jnp.int32)
    te_ref[...] = te


def _k3b(rank, cmb, off):
    return pl.pallas_call(
        _k3b_body,
        grid=(NBT,),
        in_specs=[
            pl.BlockSpec((E, BT), lambda i: (0, i)),
            pl.BlockSpec((E, BT), lambda i: (0, i)),
            pl.BlockSpec((E, 1), lambda i: (0, 0)),
        ],
        out_specs=[
            pl.BlockSpec((1, BT), lambda i: (0, i)),
            pl.BlockSpec((1, BT), lambda i: (0, i)),
            pl.BlockSpec((1, BT), lambda i: (0, i)),
            pl.BlockSpec((1, BT), lambda i: (0, i)),
            pl.BlockSpec((1, NRT), lambda i: (0, 0)),
        ],
        out_shape=[
            jax.ShapeDtypeStruct((1, T), jnp.int32),
            jax.ShapeDtypeStruct((1, T), jnp.int32),
            jax.ShapeDtypeStruct((1, T), jnp.float32),
            jax.ShapeDtypeStruct((1, T), jnp.float32),
            jax.ShapeDtypeStruct((1, NRT), jnp.int32),
        ],
    )(rank, cmb, off)


@functools.cache
def _sc_dispatch_kernel():
    mesh = plsc.VectorSubcoreMesh(core_axis_name="c", subcore_axis_name="s",
                                  num_cores=2)

    @functools.partial(
        pl.kernel, mesh=mesh,
        out_type=jax.ShapeDtypeStruct((NPAD, D), jnp.float32),
        scratch_types=[
            pltpu.VMEM((TPW, D), jnp.float32),
            pltpu.VMEM((TPW,), jnp.int32),
            pltpu.VMEM((TPW,), jnp.int32),
            pltpu.SemaphoreType.DMA,
            pltpu.SemaphoreType.DMA,
        ])
    def disp(hn2_hbm, p0_hbm, p1_hbm, x_hbm, rows_v, p0_v, p1_v, s0, s1):
        wid = jax.lax.axis_index("s") * 2 + jax.lax.axis_index("c")
        base = wid * TPW
        pltpu.sync_copy(hn2_hbm.at[pl.ds(base, TPW)], rows_v)
        pltpu.sync_copy(p0_hbm.at[pl.ds(base, TPW)], p0_v)
        pltpu.sync_copy(p1_hbm.at[pl.ds(base, TPW)], p1_v)
        c0 = pltpu.async_copy(rows_v, x_hbm.at[p0_v], s0)
        c1 = pltpu.async_copy(rows_v, x_hbm.at[p1_v], s1)
        c0.wait()
        c1.wait()

    return disp


def _sc_dispatch(hn2, p0f, p1f):
    return _sc_dispatch_kernel()(hn2, p0f, p1f)


@functools.cache
def _sc_combine_kernel():
    mesh = plsc.VectorSubcoreMesh(core_axis_name="c", subcore_axis_name="s",
                                  num_cores=2)

    @functools.partial(
        pl.kernel, mesh=mesh,
        out_type=(jax.ShapeDtypeStruct((T, D), jnp.float32),
                  jax.ShapeDtypeStruct((T, D), jnp.float32)),
        scratch_types=[
            pltpu.VMEM((TPW, D), jnp.float32),
            pltpu.VMEM((TPW,), jnp.int32),
            pltpu.VMEM((TPW,), jnp.int32),
            pltpu.SemaphoreType.DMA,
        ])
    def comb(y_hbm, p0_hbm, p1_hbm, yg0_hbm, yg1_hbm,
             rows_v, p0_v, p1_v, sem):
        wid = jax.lax.axis_index("s") * 2 + jax.lax.axis_index("c")
        base = wid * TPW
        pltpu.sync_copy(p0_hbm.at[pl.ds(base, TPW)], p0_v)
        pltpu.sync_copy(p1_hbm.at[pl.ds(base, TPW)], p1_v)
        pltpu.async_copy(y_hbm.at[p0_v], rows_v, sem).wait()
        pltpu.sync_copy(rows_v, yg0_hbm.at[pl.ds(base, TPW)])
        pltpu.async_copy(y_hbm.at[p1_v], rows_v, sem).wait()
        pltpu.sync_copy(rows_v, yg1_hbm.at[pl.ds(base, TPW)])

    return comb


def _sc_combine(y, p0f, p1f):
    return _sc_combine_kernel()(y, p0f, p1f)


def _k4_body(te_ref, x_ref, wg_ref, wu_ref, wd_ref, y_ref):
    x = x_ref[...].astype(jnp.bfloat16)
    wg = wg_ref[0].astype(jnp.bfloat16)
    wu = wu_ref[0].astype(jnp.bfloat16)
    wd = wd_ref[0].astype(jnp.bfloat16)
    g = jax.lax.dot_general(x, wg, (((1,), (1,)), ((), ())),
                            preferred_element_type=jnp.float32)
    u = jax.lax.dot_general(x, wu, (((1,), (1,)), ((), ())),
                            preferred_element_type=jnp.float32)
    a = (g * jax.nn.sigmoid(g) * u).astype(jnp.bfloat16)
    y_ref[...] = jax.lax.dot_general(a, wd, (((1,), (1,)), ((), ())),
                                     preferred_element_type=jnp.float32)


def _k4(te, x_sorted, w_gate, w_up, w_down):
    grid_spec = pltpu.PrefetchScalarGridSpec(
        num_scalar_prefetch=1,
        grid=(NRT,),
        in_specs=[
            pl.BlockSpec((BTM, D), lambda i, te_r: (i, 0)),
            pl.BlockSpec((1, DFF, D), lambda i, te_r: (te_r[i], 0, 0)),
            pl.BlockSpec((1, DFF, D), lambda i, te_r: (te_r[i], 0, 0)),
            pl.BlockSpec((1, D, DFF), lambda i, te_r: (te_r[i], 0, 0)),
        ],
        out_specs=pl.BlockSpec((BTM, D), lambda i, te_r: (i, 0)),
    )
    return pl.pallas_call(
        _k4_body,
        grid_spec=grid_spec,
        out_shape=jax.ShapeDtypeStruct((NPAD, D), jnp.float32),
    )(te, x_sorted, w_gate, w_up, w_down)


def _k5a_body(h2_ref, hn2_ref, sg_ref, su_ref, sd_ref, out_ref):
    x = hn2_ref[...].astype(jnp.bfloat16)
    g = jax.lax.dot_general(x, sg_ref[...], (((1,), (1,)), ((), ())),
                            preferred_element_type=jnp.float32)
    u = jax.lax.dot_general(x, su_ref[...], (((1,), (1,)), ((), ())),
                            preferred_element_type=jnp.float32)
    a = (g * jax.nn.sigmoid(g) * u).astype(jnp.bfloat16)
    sh = jax.lax.dot_general(a, sd_ref[...], (((1,), (1,)), ((), ())),
                             preferred_element_type=jnp.float32)
    out_ref[...] = h2_ref[...] + sh


def _k5a(h2, hn2, sg, su, sd):
    return pl.pallas_call(
        _k5a_body,
        grid=(NBT,),
        in_specs=[
            pl.BlockSpec((BT, D), lambda i: (i, 0)),
            pl.BlockSpec((BT, D), lambda i: (i, 0)),
            pl.BlockSpec((DFF, D), lambda i: (0, 0)),
            pl.BlockSpec((DFF, D), lambda i: (0, 0)),
            pl.BlockSpec((D, DFF), lambda i: (0, 0)),
        ],
        out_specs=pl.BlockSpec((BT, D), lambda i: (i, 0)),
        out_shape=jax.ShapeDtypeStruct((T, D), jnp.float32),
    )(h2, hn2, sg, su, sd)


def _k5b_body(base_ref, yg0_ref, yg1_ref, w0_ref, w1_ref, out_ref):
    out_ref[...] = (base_ref[...]
                    + yg0_ref[...] * w0_ref[...]
                    + yg1_ref[...] * w1_ref[...])


def _k5b(base, yg0, yg1, w0, w1):
    return pl.pallas_call(
        _k5b_body,
        grid=(NBT,),
        in_specs=[
            pl.BlockSpec((BT, D), lambda i: (i, 0)),
            pl.BlockSpec((BT, D), lambda i: (i, 0)),
            pl.BlockSpec((BT, D), lambda i: (i, 0)),
            pl.BlockSpec((BT, 1), lambda i: (i, 0)),
            pl.BlockSpec((BT, 1), lambda i: (i, 0)),
        ],
        out_specs=pl.BlockSpec((BT, D), lambda i: (i, 0)),
        out_shape=jax.ShapeDtypeStruct((T, D), jnp.float32),
    )(base, yg0, yg1, w0, w1)


@jax.jit
def _run(hidden_states, qkv_w, attn_gate_w, o_w, q_norm_w, k_norm_w,
         ln1_w, ln2_w, router_w, expert_bias, w_gate, w_up, w_down,
         sh_gate, sh_up, sh_down):
    h = hidden_states
    q, k, v, gate = _k1(h, ln1_w, qkv_w, attn_gate_w, q_norm_w, k_norm_w)

    mask = jnp.tril(jnp.ones((BQ, BQ), jnp.float32))
    o = _k2(q, k, v, mask)
    o2 = o.transpose(1, 0, 2).reshape(T, NH * HD)

    h2, hn2, cmb, rank, off = _k3(o2, gate, h, o_w, ln2_w, router_w,
                                  expert_bias)
    p0, p1, w0, w1, te = _k3b(rank, cmb, off)
    p0f = p0.reshape(T)
    p1f = p1.reshape(T)

    x_sorted = _sc_dispatch(hn2, p0f, p1f)
    y = _k4(te.reshape(NRT), x_sorted, w_gate, w_up, w_down)
    yg0, yg1 = _sc_combine(y, p0f, p1f)

    sg = sh_gate.astype(jnp.bfloat16)
    su = sh_up.astype(jnp.bfloat16)
    sd = sh_down.astype(jnp.bfloat16)
    base = _k5a(h2, hn2, sg, su, sd)
    return _k5b(base, yg0, yg1, w0.reshape(T, 1), w1.reshape(T, 1))


def kernel(positions, hidden_states, qkv_w, attn_gate_w, o_w, q_norm_w,
           k_norm_w, ln1_w, ln2_w, router_w, expert_bias, w_gate, w_up,
           w_down, sh_gate, sh_up, sh_down):
    return _run(hidden_states, qkv_w, attn_gate_w, o_w, q_norm_w, k_norm_w,
                ln1_w, ln2_w, router_w, expert_bias, w_gate, w_up, w_down,
                sh_gate, sh_up, sh_down)


# speedup vs baseline: 1.1942x; 1.0242x over previous
import functools
import jax
import jax.numpy as jnp
from jax.experimental import pallas as pl
from jax.experimental.pallas import tpu as pltpu
from jax.experimental.pallas import tpu_sc as plsc

T = 2048
D = 1024
NH = 16
NKV = 4
HD = 64
E = 8
TOPK = 2
NG = 4
TG = 2
DFF = 512
EPS = 1e-05

BT = 256
NBT = T // BT
BQ = 512
NBQ = T // BQ

BTM = 256
NRT = 24
NPAD = NRT * BTM
NWORK = 32
TPW = T // NWORK

_HIGH = jax.lax.Precision.HIGHEST


def _rms(x, w, eps=EPS):
    v = jnp.mean(x * x, axis=-1, keepdims=True)
    return x * jax.lax.rsqrt(v + eps) * w


def _dot_nt(a, b):
    return jax.lax.dot_general(
        a.astype(jnp.bfloat16), b.astype(jnp.bfloat16),
        (((1,), (1,)), ((), ())), preferred_element_type=jnp.float32)


def _k1_body(h_ref, ln1_ref, qkvw_ref, gatew_ref, qn_ref, kn_ref,
             q_out, k_out, v_out, gate_out):
    x = h_ref[...]
    xn = _rms(x, ln1_ref[...])
    y = _dot_nt(xn, qkvw_ref[...])
    gate_out[...] = _dot_nt(xn, gatew_ref[...]).astype(jnp.bfloat16)
    qn = qn_ref[...]
    kn = kn_ref[...]
    for h in range(NH):
        qh = _rms(y[:, h * HD:(h + 1) * HD], qn) * (HD ** -0.5)
        q_out[h] = qh.astype(jnp.bfloat16)
    for h in range(NKV):
        kb = NH * HD + h * HD
        k_out[h] = _rms(y[:, kb:kb + HD], kn).astype(jnp.bfloat16)
        vb = (NH + NKV) * HD + h * HD
        v_out[h] = y[:, vb:vb + HD].astype(jnp.bfloat16)


def _k1(h, ln1_w, qkv_w, attn_gate_w, q_norm_w, k_norm_w):
    return pl.pallas_call(
        _k1_body,
        grid=(NBT,),
        in_specs=[
            pl.BlockSpec((BT, D), lambda i: (i, 0)),
            pl.BlockSpec((1, D), lambda i: (0, 0)),
            pl.BlockSpec(((NH + 2 * NKV) * HD, D), lambda i: (0, 0)),
            pl.BlockSpec((NH * HD, D), lambda i: (0, 0)),
            pl.BlockSpec((1, HD), lambda i: (0, 0)),
            pl.BlockSpec((1, HD), lambda i: (0, 0)),
        ],
        out_specs=[
            pl.BlockSpec((NH, BT, HD), lambda i: (0, i, 0)),
            pl.BlockSpec((NKV, BT, HD), lambda i: (0, i, 0)),
            pl.BlockSpec((NKV, BT, HD), lambda i: (0, i, 0)),
            pl.BlockSpec((BT, NH * HD), lambda i: (i, 0)),
        ],
        out_shape=[
            jax.ShapeDtypeStruct((NH, T, HD), jnp.bfloat16),
            jax.ShapeDtypeStruct((NKV, T, HD), jnp.bfloat16),
            jax.ShapeDtypeStruct((NKV, T, HD), jnp.bfloat16),
            jax.ShapeDtypeStruct((T, NH * HD), jnp.bfloat16),
        ],
    )(h, ln1_w.reshape(1, D), qkv_w, attn_gate_w,
      q_norm_w.reshape(1, HD), k_norm_w.reshape(1, HD))


def _attn_body(q_ref, k_ref, v_ref, mask_ref, o_ref):
    i = pl.program_id(1)
    hpg = NH // NKV
    qs = [q_ref[j] for j in range(hpg)]

    def pexp(s):
        return jnp.exp(jnp.minimum(s, 30.0))

    def chunk(kb, vb, carry, maskf=None):
        ls, accs = carry
        ss = [jax.lax.dot_general(qj, kb, (((1,), (1,)), ((), ())),
                                  preferred_element_type=jnp.float32)
              for qj in qs]
        ps = [pexp(s) for s in ss]
        if maskf is not None:
            ps = [p * maskf for p in ps]
        ls = tuple(l + jnp.sum(p, axis=-1, keepdims=True)
                   for l, p in zip(ls, ps))
        accs = tuple(acc + jax.lax.dot_general(
            p.astype(jnp.bfloat16), vb, (((1,), (0,)), ((), ())),
            preferred_element_type=jnp.float32)
            for acc, p in zip(accs, ps))
        return ls, accs

    def body(j, carry):
        kb = k_ref[0, pl.ds(j * BQ, BQ), :]
        vb = v_ref[0, pl.ds(j * BQ, BQ), :]
        return chunk(kb, vb, carry)

    z1 = jnp.zeros((BQ, 1), jnp.float32)
    za = jnp.zeros((BQ, HD), jnp.float32)
    carry = ((z1,) * hpg, (za,) * hpg)
    carry = jax.lax.fori_loop(0, i, body, carry)

    kb = k_ref[0, pl.ds(i * BQ, BQ), :]
    vb = v_ref[0, pl.ds(i * BQ, BQ), :]
    ls, accs = chunk(kb, vb, carry, mask_ref[...])

    for j in range(hpg):
        o_ref[j] = (accs[j] / ls[j]).astype(jnp.bfloat16)


def _k2(q, k, v, mask):
    rep = NH // NKV
    return pl.pallas_call(
        _attn_body,
        grid=(NKV, NBQ),
        in_specs=[
            pl.BlockSpec((rep, BQ, HD), lambda h, i: (h, i, 0)),
            pl.BlockSpec((1, T, HD), lambda h, i: (h, 0, 0)),
            pl.BlockSpec((1, T, HD), lambda h, i: (h, 0, 0)),
            pl.BlockSpec((BQ, BQ), lambda h, i: (0, 0)),
        ],
        out_specs=pl.BlockSpec((rep, BQ, HD), lambda h, i: (h, i, 0)),
        out_shape=jax.ShapeDtypeStruct((NH, T, HD), jnp.bfloat16),
    )(q, k, v, mask)


def _rank_lt_t(m, kmax, n):
    rank = jnp.zeros_like(m)
    e_idx = jax.lax.broadcasted_iota(jnp.int32, m.shape, 0)
    for r in range(1, n):
        m_rot = jnp.concatenate([m[r:, :], m[:r, :]], axis=0)
        beat = (m_rot > m) | ((m_rot == m) & (e_idx >= n - r))
        rank = rank + beat.astype(jnp.float32)
    return (rank < kmax).astype(jnp.float32)


def _dot_high(a, b):
    return jax.lax.dot_general(a, b, (((1,), (0,)), ((), ())),
                               preferred_element_type=jnp.float32,
                               precision=_HIGH)


def _lt8():
    ea = jax.lax.broadcasted_iota(jnp.int32, (E, E), 0)
    eb = jax.lax.broadcasted_iota(jnp.int32, (E, E), 1)
    return (eb < ea).astype(jnp.float32)


def _k3_body(o_ref, gate_ref, res_ref, ow_ref, ln2_ref, rw_ref, bias_ref,
             h2_out, hn2_out, cmb_out, rank_out, off_out, cnt_ref):
    i = pl.program_id(0)

    @pl.when(i == 0)
    def _init():
        cnt_ref[...] = jnp.zeros((E, 1), jnp.float32)

    og = o_ref[...] * jax.nn.sigmoid(gate_ref[...])
    h2 = _dot_nt(og, ow_ref[...]) + res_ref[...]
    h2_out[...] = h2
    hn2 = _rms(h2, ln2_ref[...])
    hn2_out[...] = hn2

    logits = jax.lax.dot_general(
        rw_ref[...], hn2, (((1,), (1,)), ((), ())),
        preferred_element_type=jnp.float32, precision=_HIGH)
    scores = jax.nn.sigmoid(logits)
    sfc = scores + bias_ref[...]

    epg = E // NG
    pg = jax.lax.broadcasted_iota(jnp.int32, (NG, E), 0)
    pe = jax.lax.broadcasted_iota(jnp.int32, (NG, E), 1)
    pair = ((pe // epg) == pg).astype(jnp.float32)
    gs = _dot_high(pair, sfc)

    sel_g = _rank_lt_t(gs, TG, NG)
    ee = jax.lax.broadcasted_iota(jnp.int32, (E, NG), 0)
    gg = jax.lax.broadcasted_iota(jnp.int32, (E, NG), 1)
    expand = (gg == (ee // epg)).astype(jnp.float32)
    mask_e = _dot_high(expand, sel_g)

    masked = jnp.where(mask_e > 0.5, sfc, -1e30)
    sel_e = _rank_lt_t(masked, TOPK, E)
    w = scores * sel_e
    denom = jnp.sum(w, axis=0, keepdims=True) + 1e-20
    cmb_out[...] = w / denom

    lr = jax.lax.broadcasted_iota(jnp.int32, (BT, BT), 0)
    lc = jax.lax.broadcasted_iota(jnp.int32, (BT, BT), 1)
    ltl = (lr < lc).astype(jnp.float32)
    rank_blk = _dot_high(sel_e, ltl)
    rank_out[...] = rank_blk + cnt_ref[...]
    new_cnt = cnt_ref[...] + jnp.sum(sel_e, axis=1, keepdims=True)
    cnt_ref[...] = new_cnt
    padded = jnp.floor((new_cnt + (BTM - 1)) * (1.0 / BTM)) * BTM
    off_out[...] = _dot_high(_lt8(), padded)


def _k3(o2, gate, res, o_w, ln2_w, router_w, expert_bias):
    return pl.pallas_call(
        _k3_body,
        grid=(NBT,),
        in_specs=[
            pl.BlockSpec((BT, D), lambda i: (i, 0)),
            pl.BlockSpec((BT, D), lambda i: (i, 0)),
            pl.BlockSpec((BT, D), lambda i: (i, 0)),
            pl.BlockSpec((D, D), lambda i: (0, 0)),
            pl.BlockSpec((1, D), lambda i: (0, 0)),
            pl.BlockSpec((E, D), lambda i: (0, 0)),
            pl.BlockSpec((E, 1), lambda i: (0, 0)),
        ],
        out_specs=[
            pl.BlockSpec((BT, D), lambda i: (i, 0)),
            pl.BlockSpec((BT, D), lambda i: (i, 0)),
            pl.BlockSpec((E, BT), lambda i: (0, i)),
            pl.BlockSpec((E, BT), lambda i: (0, i)),
            pl.BlockSpec((E, 1), lambda i: (0, 0)),
        ],
        out_shape=[
            jax.ShapeDtypeStruct((T, D), jnp.float32),
            jax.ShapeDtypeStruct((T, D), jnp.float32),
            jax.ShapeDtypeStruct((E, T), jnp.float32),
            jax.ShapeDtypeStruct((E, T), jnp.float32),
            jax.ShapeDtypeStruct((E, 1), jnp.float32),
        ],
        scratch_shapes=[pltpu.VMEM((E, 1), jnp.float32)],
    )(o2, gate, res, o_w, ln2_w.reshape(1, D), router_w,
      expert_bias.reshape(E, 1))


def _k3b_body(rank_ref, cmb_ref, off_ref, p0_ref, p1_ref, w0_ref, w1_ref,
              te_ref):
    cmb = cmb_ref[...]
    sel = (cmb > 0).astype(jnp.float32)
    s_excl = _dot_high(_lt8(), sel)
    off = off_ref[...]
    pos = off + rank_ref[...]
    is0 = sel * (s_excl == 0).astype(jnp.float32)
    is1 = sel * (s_excl == 1).astype(jnp.float32)
    p0_ref[...] = jnp.sum(pos * is0, axis=0, keepdims=True).astype(jnp.int32)
    p1_ref[...] = jnp.sum(pos * is1, axis=0, keepdims=True).astype(jnp.int32)
    w0_ref[...] = jnp.sum(cmb * is0, axis=0, keepdims=True)
    w1_ref[...] = jnp.sum(cmb * is1, axis=0, keepdims=True)

    jt = jax.lax.broadcasted_iota(jnp.int32, (1, NRT), 1) * BTM
    te = jnp.zeros((1, NRT), jnp.int32)
    for e in range(1, E):
        te = te + (jt >= off[e:e + 1, :].astype(jnp.int32)).astype(jnp.int32)
    te_ref[...] = te


def _k3b(rank, cmb, off):
    return pl.pallas_call(
        _k3b_body,
        grid=(NBT,),
        in_specs=[
            pl.BlockSpec((E, BT), lambda i: (0, i)),
            pl.BlockSpec((E, BT), lambda i: (0, i)),
            pl.BlockSpec((E, 1), lambda i: (0, 0)),
        ],
        out_specs=[
            pl.BlockSpec((1, BT), lambda i: (0, i)),
            pl.BlockSpec((1, BT), lambda i: (0, i)),
            pl.BlockSpec((1, BT), lambda i: (0, i)),
            pl.BlockSpec((1, BT), lambda i: (0, i)),
            pl.BlockSpec((1, NRT), lambda i: (0, 0)),
        ],
        out_shape=[
            jax.ShapeDtypeStruct((1, T), jnp.int32),
            jax.ShapeDtypeStruct((1, T), jnp.int32),
            jax.ShapeDtypeStruct((1, T), jnp.float32),
            jax.ShapeDtypeStruct((1, T), jnp.float32),
            jax.ShapeDtypeStruct((1, NRT), jnp.int32),
        ],
    )(rank, cmb, off)


@functools.cache
def _sc_dispatch_kernel():
    mesh = plsc.VectorSubcoreMesh(core_axis_name="c", subcore_axis_name="s",
                                  num_cores=2)

    @functools.partial(
        pl.kernel, mesh=mesh,
        out_type=jax.ShapeDtypeStruct((NPAD, D), jnp.float32),
        scratch_types=[
            pltpu.VMEM((TPW, D), jnp.float32),
            pltpu.VMEM((TPW,), jnp.int32),
            pltpu.VMEM((TPW,), jnp.int32),
            pltpu.SemaphoreType.DMA,
            pltpu.SemaphoreType.DMA,
        ])
    def disp(hn2_hbm, p0_hbm, p1_hbm, x_hbm, rows_v, p0_v, p1_v, s0, s1):
        wid = jax.lax.axis_index("s") * 2 + jax.lax.axis_index("c")
        base = wid * TPW
        pltpu.sync_copy(hn2_hbm.at[pl.ds(base, TPW)], rows_v)
        pltpu.sync_copy(p0_hbm.at[pl.ds(base, TPW)], p0_v)
        pltpu.sync_copy(p1_hbm.at[pl.ds(base, TPW)], p1_v)
        c0 = pltpu.async_copy(rows_v, x_hbm.at[p0_v], s0)
        c1 = pltpu.async_copy(rows_v, x_hbm.at[p1_v], s1)
        c0.wait()
        c1.wait()

    return disp


def _sc_dispatch(hn2, p0f, p1f):
    return _sc_dispatch_kernel()(hn2, p0f, p1f)


@functools.cache
def _sc_combine_kernel():
    mesh = plsc.VectorSubcoreMesh(core_axis_name="c", subcore_axis_name="s",
                                  num_cores=2)

    @functools.partial(
        pl.kernel, mesh=mesh,
        out_type=(jax.ShapeDtypeStruct((T, D), jnp.float32),
                  jax.ShapeDtypeStruct((T, D), jnp.float32)),
        scratch_types=[
            pltpu.VMEM((TPW, D), jnp.float32),
            pltpu.VMEM((TPW,), jnp.int32),
            pltpu.VMEM((TPW,), jnp.int32),
            pltpu.SemaphoreType.DMA,
        ])
    def comb(y_hbm, p0_hbm, p1_hbm, yg0_hbm, yg1_hbm,
             rows_v, p0_v, p1_v, sem):
        wid = jax.lax.axis_index("s") * 2 + jax.lax.axis_index("c")
        base = wid * TPW
        pltpu.sync_copy(p0_hbm.at[pl.ds(base, TPW)], p0_v)
        pltpu.sync_copy(p1_hbm.at[pl.ds(base, TPW)], p1_v)
        pltpu.async_copy(y_hbm.at[p0_v], rows_v, sem).wait()
        pltpu.sync_copy(rows_v, yg0_hbm.at[pl.ds(base, TPW)])
        pltpu.async_copy(y_hbm.at[p1_v], rows_v, sem).wait()
        pltpu.sync_copy(rows_v, yg1_hbm.at[pl.ds(base, TPW)])

    return comb


def _sc_combine(y, p0f, p1f):
    return _sc_combine_kernel()(y, p0f, p1f)


def _k4_body(te_ref, x_ref, wg_ref, wu_ref, wd_ref, y_ref):
    x = x_ref[...].astype(jnp.bfloat16)
    wg = wg_ref[0].astype(jnp.bfloat16)
    wu = wu_ref[0].astype(jnp.bfloat16)
    wd = wd_ref[0].astype(jnp.bfloat16)
    g = jax.lax.dot_general(x, wg, (((1,), (1,)), ((), ())),
                            preferred_element_type=jnp.float32)
    u = jax.lax.dot_general(x, wu, (((1,), (1,)), ((), ())),
                            preferred_element_type=jnp.float32)
    a = (g * jax.nn.sigmoid(g) * u).astype(jnp.bfloat16)
    y_ref[...] = jax.lax.dot_general(a, wd, (((1,), (1,)), ((), ())),
                                     preferred_element_type=jnp.float32)


def _k4(te, x_sorted, w_gate, w_up, w_down):
    grid_spec = pltpu.PrefetchScalarGridSpec(
        num_scalar_prefetch=1,
        grid=(NRT,),
        in_specs=[
            pl.BlockSpec((BTM, D), lambda i, te_r: (i, 0)),
            pl.BlockSpec((1, DFF, D), lambda i, te_r: (te_r[i], 0, 0)),
            pl.BlockSpec((1, DFF, D), lambda i, te_r: (te_r[i], 0, 0)),
            pl.BlockSpec((1, D, DFF), lambda i, te_r: (te_r[i], 0, 0)),
        ],
        out_specs=pl.BlockSpec((BTM, D), lambda i, te_r: (i, 0)),
    )
    return pl.pallas_call(
        _k4_body,
        grid_spec=grid_spec,
        out_shape=jax.ShapeDtypeStruct((NPAD, D), jnp.float32),
    )(te, x_sorted, w_gate, w_up, w_down)


def _k5_body(h2_ref, hn2_ref, yg0_ref, yg1_ref, w0_ref, w1_ref,
             sg_ref, su_ref, sd_ref, out_ref):
    x = hn2_ref[...].astype(jnp.bfloat16)
    g = jax.lax.dot_general(x, sg_ref[...], (((1,), (1,)), ((), ())),
                            preferred_element_type=jnp.float32)
    u = jax.lax.dot_general(x, su_ref[...], (((1,), (1,)), ((), ())),
                            preferred_element_type=jnp.float32)
    a = (g * jax.nn.sigmoid(g) * u).astype(jnp.bfloat16)
    sh = jax.lax.dot_general(a, sd_ref[...], (((1,), (1,)), ((), ())),
                             preferred_element_type=jnp.float32)
    out_ref[...] = (h2_ref[...] + sh
                    + yg0_ref[...] * w0_ref[...]
                    + yg1_ref[...] * w1_ref[...])


def _k5(h2, hn2, yg0, yg1, w0, w1, sg, su, sd):
    return pl.pallas_call(
        _k5_body,
        grid=(NBT,),
        in_specs=[
            pl.BlockSpec((BT, D), lambda i: (i, 0)),
            pl.BlockSpec((BT, D), lambda i: (i, 0)),
            pl.BlockSpec((BT, D), lambda i: (i, 0)),
            pl.BlockSpec((BT, D), lambda i: (i, 0)),
            pl.BlockSpec((BT, 1), lambda i: (i, 0)),
            pl.BlockSpec((BT, 1), lambda i: (i, 0)),
            pl.BlockSpec((DFF, D), lambda i: (0, 0)),
            pl.BlockSpec((DFF, D), lambda i: (0, 0)),
            pl.BlockSpec((D, DFF), lambda i: (0, 0)),
        ],
        out_specs=pl.BlockSpec((BT, D), lambda i: (i, 0)),
        out_shape=jax.ShapeDtypeStruct((T, D), jnp.float32),
    )(h2, hn2, yg0, yg1, w0, w1, sg, su, sd)


@jax.jit
def _run(hidden_states, qkv_w, attn_gate_w, o_w, q_norm_w, k_norm_w,
         ln1_w, ln2_w, router_w, expert_bias, w_gate, w_up, w_down,
         sh_gate, sh_up, sh_down):
    h = hidden_states
    q, k, v, gate = _k1(h, ln1_w, qkv_w, attn_gate_w, q_norm_w, k_norm_w)

    mask = jnp.tril(jnp.ones((BQ, BQ), jnp.float32))
    o = _k2(q, k, v, mask)
    o2 = o.transpose(1, 0, 2).reshape(T, NH * HD)

    h2, hn2, cmb, rank, off = _k3(o2, gate, h, o_w, ln2_w, router_w,
                                  expert_bias)
    p0, p1, w0, w1, te = _k3b(rank, cmb, off)
    p0f = p0.reshape(T)
    p1f = p1.reshape(T)

    x_sorted = _sc_dispatch(hn2, p0f, p1f)
    y = _k4(te.reshape(NRT), x_sorted, w_gate, w_up, w_down)
    yg0, yg1 = _sc_combine(y, p0f, p1f)

    sg = sh_gate.astype(jnp.bfloat16)
    su = sh_up.astype(jnp.bfloat16)
    sd = sh_down.astype(jnp.bfloat16)
    return _k5(h2, hn2, yg0, yg1, w0.reshape(T, 1), w1.reshape(T, 1),
               sg, su, sd)


def kernel(positions, hidden_states, qkv_w, attn_gate_w, o_w, q_norm_w,
           k_norm_w, ln1_w, ln2_w, router_w, expert_bias, w_gate, w_up,
           w_down, sh_gate, sh_up, sh_down):
    return _run(hidden_states, qkv_w, attn_gate_w, o_w, q_norm_w, k_norm_w,
                ln1_w, ln2_w, router_w, expert_bias, w_gate, w_up, w_down,
                sh_gate, sh_up, sh_down)
